# baseline (speedup 1.0000x reference)
import functools

import jax
import jax.numpy as jnp
from jax import lax
from jax.experimental import pallas as pl
from jax.experimental.pallas import tpu as pltpu
from jax.experimental.pallas import tpu_sc as plsc

N_NODES = 10000
N_EDGES = 320000
EPS = 1e-5

NC = 2
NS = 16
NW = NC * NS
CHUNK = 80
CNT_COL = 64

f32 = jnp.float32

_SC_MESH = dict(core_axis_name="c", subcore_axis_name="s")


_R_EDGE = 2560
_R_NODE = 2000

_ARB = pltpu.CompilerParams(dimension_semantics=("arbitrary",))


def _uv_from_x(x, Wd, Wb, b1):
    n, d = x.shape
    h = Wd.shape[1]

    def kern(x_ref, wd_ref, wb_ref, b1_ref, u_ref, v_ref):
        y = x_ref[...]
        u_ref[...] = jnp.dot(y, wd_ref[...], preferred_element_type=f32) + b1_ref[...]
        v_ref[...] = jnp.dot(y, wb_ref[...], preferred_element_type=f32)

    return pl.pallas_call(
        kern,
        grid=(n // _R_NODE,),
        in_specs=[
            pl.BlockSpec((_R_NODE, d), lambda i: (i, 0)),
            pl.BlockSpec((d, h), lambda i: (0, 0)),
            pl.BlockSpec((d, h), lambda i: (0, 0)),
            pl.BlockSpec((1, h), lambda i: (0, 0)),
        ],
        out_specs=[
            pl.BlockSpec((_R_NODE, h), lambda i: (i, 0)),
            pl.BlockSpec((_R_NODE, h), lambda i: (i, 0)),
        ],
        out_shape=[
            jax.ShapeDtypeStruct((n, h), f32),
            jax.ShapeDtypeStruct((n, h), f32),
        ],
        compiler_params=_ARB,
    )(x, Wd, Wb, b1.reshape(1, h))


def _uv_sum_halves(s_arr, cnt_arr, Wd, Wb, b1, d_use):
    _, n, w = s_arr.shape
    h = Wd.shape[1]

    def kern(s_ref, cnt_ref, wd_ref, wb_ref, b1_ref, u_ref, v_ref):
        cnt = (cnt_ref[0, :, CNT_COL:CNT_COL + 1]
               + cnt_ref[1, :, CNT_COL:CNT_COL + 1])
        inv = 1.0 / jnp.maximum(cnt, 1.0)
        tot = s_ref[0] + s_ref[1]
        y = tot[:, :d_use] * inv
        u_ref[...] = jnp.dot(y, wd_ref[...], preferred_element_type=f32) + b1_ref[...]
        v_ref[...] = jnp.dot(y, wb_ref[...], preferred_element_type=f32)

    return pl.pallas_call(
        kern,
        grid=(n // _R_NODE,),
        in_specs=[
            pl.BlockSpec((2, _R_NODE, w), lambda i: (0, i, 0)),
            pl.BlockSpec((2, _R_NODE, 128), lambda i: (0, i, 0)),
            pl.BlockSpec((d_use, h), lambda i: (0, 0)),
            pl.BlockSpec((d_use, h), lambda i: (0, 0)),
            pl.BlockSpec((1, h), lambda i: (0, 0)),
        ],
        out_specs=[
            pl.BlockSpec((_R_NODE, h), lambda i: (i, 0)),
            pl.BlockSpec((_R_NODE, h), lambda i: (i, 0)),
        ],
        out_shape=[
            jax.ShapeDtypeStruct((n, h), f32),
            jax.ShapeDtypeStruct((n, h), f32),
        ],
        compiler_params=_ARB,
    )(s_arr, cnt_arr, Wd, Wb, b1.reshape(1, h))


def _stats(h1):
    e, h = h1.shape

    def kern(h_ref, s_ref, q_ref):
        @pl.when(pl.program_id(0) == 0)
        def _():
            s_ref[...] = jnp.zeros_like(s_ref)
            q_ref[...] = jnp.zeros_like(q_ref)

        v = h_ref[...]
        s_ref[...] += jnp.sum(v, axis=0, keepdims=True)
        q_ref[...] += jnp.sum(v * v, axis=0, keepdims=True)

    return pl.pallas_call(
        kern,
        grid=(e // _R_EDGE,),
        in_specs=[pl.BlockSpec((_R_EDGE, h), lambda i: (i, 0))],
        out_specs=[
            pl.BlockSpec((1, h), lambda i: (0, 0)),
            pl.BlockSpec((1, h), lambda i: (0, 0)),
        ],
        out_shape=[
            jax.ShapeDtypeStruct((1, h), f32),
            jax.ShapeDtypeStruct((1, h), f32),
        ],
        compiler_params=_ARB,
    )(h1)


def _mlp_pass(h_in, a, c, W, b):
    e, hi = h_in.shape
    ho = W.shape[1]

    def kern(h_ref, a_ref, c_ref, w_ref, b_ref, o_ref, s_ref, q_ref):
        @pl.when(pl.program_id(0) == 0)
        def _():
            s_ref[...] = jnp.zeros_like(s_ref)
            q_ref[...] = jnp.zeros_like(q_ref)

        m = jnp.maximum(h_ref[...] * a_ref[...] + c_ref[...], 0.0)
        v = jnp.dot(m.astype(jnp.bfloat16), w_ref[...].astype(jnp.bfloat16),
                    preferred_element_type=f32) + b_ref[...]
        o_ref[...] = v
        s_ref[...] += jnp.sum(v, axis=0, keepdims=True)
        q_ref[...] += jnp.sum(v * v, axis=0, keepdims=True)

    return pl.pallas_call(
        kern,
        grid=(e // _R_EDGE,),
        in_specs=[
            pl.BlockSpec((_R_EDGE, hi), lambda i: (i, 0)),
            pl.BlockSpec((1, hi), lambda i: (0, 0)),
            pl.BlockSpec((1, hi), lambda i: (0, 0)),
            pl.BlockSpec((hi, ho), lambda i: (0, 0)),
            pl.BlockSpec((1, ho), lambda i: (0, 0)),
        ],
        out_specs=[
            pl.BlockSpec((_R_EDGE, ho), lambda i: (i, 0)),
            pl.BlockSpec((1, ho), lambda i: (0, 0)),
            pl.BlockSpec((1, ho), lambda i: (0, 0)),
        ],
        out_shape=[
            jax.ShapeDtypeStruct((e, ho), f32),
            jax.ShapeDtypeStruct((1, ho), f32),
            jax.ShapeDtypeStruct((1, ho), f32),
        ],
        compiler_params=_ARB,
    )(h_in, a, c, W, b.reshape(1, ho))


def _affine_pad(h3, a, c, wout, cnt_col):
    e, h = h3.shape
    extra = wout - h

    def kern(h_ref, a_ref, c_ref, o_ref):
        m = jnp.maximum(h_ref[...] * a_ref[...] + c_ref[...], 0.0)
        if extra == 0:
            o_ref[...] = m
        else:
            if cnt_col:
                col = lax.broadcasted_iota(jnp.int32, (_R_EDGE, extra), 1)
                pad = jnp.where(col == 0, 1.0, 0.0).astype(f32)
            else:
                pad = jnp.zeros((_R_EDGE, extra), f32)
            o_ref[...] = jnp.concatenate([m, pad], axis=-1)

    return pl.pallas_call(
        kern,
        grid=(e // _R_EDGE,),
        in_specs=[
            pl.BlockSpec((_R_EDGE, h), lambda i: (i, 0)),
            pl.BlockSpec((1, h), lambda i: (0, 0)),
            pl.BlockSpec((1, h), lambda i: (0, 0)),
        ],
        out_specs=[pl.BlockSpec((_R_EDGE, wout), lambda i: (i, 0))],
        out_shape=[jax.ShapeDtypeStruct((e, wout), f32)],
        compiler_params=_ARB,
    )(h3, a, c)[0]


def _affine_split(h3, a, c):
    e, h = h3.shape
    h2 = h // 2

    def kern(h_ref, a_ref, c_ref, o_ref):
        m = jnp.maximum(h_ref[...] * a_ref[...] + c_ref[...], 0.0)
        o_ref[0] = m[:, :h2]
        o_ref[1] = m[:, h2:]

    return pl.pallas_call(
        kern,
        grid=(e // _R_EDGE,),
        in_specs=[
            pl.BlockSpec((_R_EDGE, h), lambda i: (i, 0)),
            pl.BlockSpec((1, h), lambda i: (0, 0)),
            pl.BlockSpec((1, h), lambda i: (0, 0)),
        ],
        out_specs=[pl.BlockSpec((2, _R_EDGE, h2), lambda i: (0, i, 0))],
        out_shape=[jax.ShapeDtypeStruct((2, e, h2), f32)],
        compiler_params=_ARB,
    )(h3, a, c)[0]


def _head(s_arr, cnt_arr, Wl, bl):
    _, n, hp2 = s_arr.shape
    d = 2 * hp2

    def kern(s_ref, cnt_ref, w_ref, b_ref, o_ref):
        cnt = (cnt_ref[0, :, CNT_COL:CNT_COL + 1]
               + cnt_ref[1, :, CNT_COL:CNT_COL + 1])
        inv = 1.0 / jnp.maximum(cnt, 1.0)
        y = jnp.concatenate([s_ref[0], s_ref[1]], axis=-1) * inv
        z = jnp.dot(y, w_ref[...], preferred_element_type=f32) + b_ref[...]
        o_ref[...] = jax.nn.sigmoid(z)

    return pl.pallas_call(
        kern,
        grid=(n // _R_NODE,),
        in_specs=[
            pl.BlockSpec((2, _R_NODE, hp2), lambda i: (0, i, 0)),
            pl.BlockSpec((2, _R_NODE, 128), lambda i: (0, i, 0)),
            pl.BlockSpec((d, 1), lambda i: (0, 0)),
            pl.BlockSpec((1, 1), lambda i: (0, 0)),
        ],
        out_specs=[pl.BlockSpec((_R_NODE, 1), lambda i: (i, 0))],
        out_shape=[jax.ShapeDtypeStruct((n, 1), f32)],
        compiler_params=_ARB,
    )(s_arr, cnt_arr, Wl, bl.reshape(1, 1))[0]


def _sc_gather(U, V, dst, src):
    n, h = U.shape
    hf = h // 16
    ept = N_EDGES // NW
    nch = ept // CHUNK

    mesh = plsc.VectorSubcoreMesh(**_SC_MESH)

    @functools.partial(
        pl.kernel,
        mesh=mesh,
        out_type=jax.ShapeDtypeStruct((N_EDGES, h), f32),
        scratch_types=[
            pltpu.VMEM((CHUNK,), jnp.int32),
            pltpu.VMEM((CHUNK,), jnp.int32),
            pltpu.VMEM((CHUNK,), jnp.int32),
            pltpu.VMEM((CHUNK,), jnp.int32),
            pltpu.VMEM((CHUNK, h), f32),
            pltpu.VMEM((CHUNK, h), f32),
            pltpu.VMEM((CHUNK, h), f32),
            pltpu.VMEM((CHUNK, h), f32),
            pltpu.SemaphoreType.DMA,
            pltpu.SemaphoreType.DMA,
            pltpu.SemaphoreType.DMA,
            pltpu.SemaphoreType.DMA,
            pltpu.SemaphoreType.DMA,
            pltpu.SemaphoreType.DMA,
        ],
    )
    def k(u_hbm, v_hbm, dst_hbm, src_hbm, h1_hbm,
          id0, is0, id1, is1, ra0, rb0, ra1, rb1,
          semi0, semi1, semr0, semr1, semw0, semw1):
        cid = lax.axis_index("c")
        sid = lax.axis_index("s")
        wid = sid * NC + cid
        base = wid * ept

        def off(c):
            return pl.multiple_of(base + c * CHUNK, 8)

        def issue_idx(c, id_, is_, sem):
            pltpu.async_copy(dst_hbm.at[pl.ds(off(c), CHUNK)], id_, sem)
            pltpu.async_copy(src_hbm.at[pl.ds(off(c), CHUNK)], is_, sem)

        def wait_idx(id_, is_, sem):
            pltpu.make_async_copy(dst_hbm.at[pl.ds(0, CHUNK)], id_, sem).wait()
            pltpu.make_async_copy(src_hbm.at[pl.ds(0, CHUNK)], is_, sem).wait()

        def issue_gather(id_, is_, ra_, rb_, sem):
            pltpu.async_copy(u_hbm.at[id_], ra_, sem)
            pltpu.async_copy(v_hbm.at[is_], rb_, sem)

        def wait_gather(id_, is_, ra_, rb_, sem):
            pltpu.make_async_copy(u_hbm.at[id_], ra_, sem).wait()
            pltpu.make_async_copy(v_hbm.at[is_], rb_, sem).wait()

        def compute(ra_, rb_):
            def rowfn(r, carry2):
                for f in range(hf):
                    sl = pl.ds(f * 16, 16)
                    plsc.addupdate(ra_.at[r, sl], rb_[r, sl])
                return carry2

            lax.fori_loop(0, CHUNK, rowfn, 0)

        def issue_write(c, ra_, sem):
            pltpu.async_copy(ra_, h1_hbm.at[pl.ds(off(c), CHUNK)], sem)

        def wait_write(ra_, sem):
            pltpu.make_async_copy(ra_, h1_hbm.at[pl.ds(0, CHUNK)], sem).wait()

        issue_idx(0, id0, is0, semi0)
        wait_idx(id0, is0, semi0)
        issue_gather(id0, is0, ra0, rb0, semr0)
        issue_idx(1, id1, is1, semi1)

        def pairbody(kp, carry):
            c1 = 2 * kp + 1
            c2 = 2 * kp + 2
            c3 = 2 * kp + 3
            wait_gather(id0, is0, ra0, rb0, semr0)
            compute(ra0, rb0)
            issue_write(2 * kp, ra0, semw0)

            @pl.when(c1 < nch)
            def _():
                wait_idx(id1, is1, semi1)

                @pl.when(kp > 0)
                def _():
                    wait_write(ra1, semw1)

                issue_gather(id1, is1, ra1, rb1, semr1)

                @pl.when(c2 < nch)
                def _():
                    issue_idx(c2, id0, is0, semi0)

            @pl.when(c1 < nch)
            def _():
                wait_gather(id1, is1, ra1, rb1, semr1)
                compute(ra1, rb1)
                issue_write(c1, ra1, semw1)

                @pl.when(c2 < nch)
                def _():
                    wait_idx(id0, is0, semi0)
                    wait_write(ra0, semw0)
                    issue_gather(id0, is0, ra0, rb0, semr0)

                    @pl.when(c3 < nch)
                    def _():
                        issue_idx(c3, id1, is1, semi1)

            return carry

        lax.fori_loop(0, (nch + 1) // 2, pairbody, 0)
        wait_write(ra0, semw0)
        wait_write(ra1, semw1)

    return k(U, V, dst, src)


def _sc_scatter(msg, dst, edge_split):
    if edge_split:
        ept = N_EDGES // 2 // NS
    else:
        ept = N_EDGES // NS
    nch = ept // CHUNK
    w = 128
    zrows = 80
    nblk = N_NODES // zrows

    mesh = plsc.VectorSubcoreMesh(**_SC_MESH)

    @functools.partial(
        pl.kernel,
        mesh=mesh,
        out_type=jax.ShapeDtypeStruct((NC, N_NODES, w), f32),
        scratch_types=[
            pltpu.VMEM((CHUNK,), jnp.int32),
            pltpu.VMEM((CHUNK,), jnp.int32),
            pltpu.VMEM((CHUNK, w), f32),
            pltpu.VMEM((CHUNK, w), f32),
            pltpu.VMEM((zrows, w), f32),
            pltpu.VMEM_SHARED((N_NODES, w), f32),
            pltpu.SemaphoreType.DMA,
            pltpu.SemaphoreType.DMA,
            pltpu.SemaphoreType.DMA,
            pltpu.SemaphoreType.DMA,
        ],
    )
    def k(msg_hbm, dst_hbm, s_hbm, ib0, ib1, bf0, bf1, zbuf, acc_sh,
          semL0, semL1, semS0, semS1):
        cid = lax.axis_index("c")
        sid = lax.axis_index("s")
        zero = jnp.zeros((16,), f32)

        def zf(r, carry):
            for f in range(w // 16):
                zbuf[r, pl.ds(f * 16, 16)] = zero
            return carry

        lax.fori_loop(0, zrows, zf, 0)
        nb = jnp.where(sid < nblk - (nblk // NS) * NS, nblk // NS + 1, nblk // NS)

        def zcopy(j, carry):
            off = pl.multiple_of((sid + j * NS) * zrows, 8)
            pltpu.sync_copy(zbuf, acc_sh.at[pl.ds(off, zrows)])
            return carry

        lax.fori_loop(0, nb, zcopy, 0)
        plsc.subcore_barrier()

        if edge_split:
            base = cid * (N_EDGES // 2) + sid * ept
        else:
            base = sid * ept

        def off(c):
            return pl.multiple_of(base + c * CHUNK, 8)

        def msg_slice(c):
            if edge_split:
                return msg_hbm.at[pl.ds(off(c), CHUNK)]
            return msg_hbm.at[cid, pl.ds(off(c), CHUNK)]

        def issue_load(c, ib, bf, sem):
            pltpu.async_copy(dst_hbm.at[pl.ds(off(c), CHUNK)], ib, sem)
            pltpu.async_copy(msg_slice(c), bf, sem)

        def wait_load(ib, bf, sem):
            pltpu.make_async_copy(dst_hbm.at[pl.ds(0, CHUNK)], ib, sem).wait()
            pltpu.make_async_copy(msg_slice(0), bf, sem).wait()

        def issue_scat(ib, bf, sem):
            pltpu.async_copy(bf, acc_sh.at[ib], sem, add=True)

        def wait_scat(ib, bf, sem):
            pltpu.make_async_copy(bf, acc_sh.at[ib], sem).wait()

        issue_load(0, ib0, bf0, semL0)
        issue_load(1, ib1, bf1, semL1)

        def pairbody(kp, carry):
            c1 = 2 * kp + 1
            c2 = 2 * kp + 2
            c3 = 2 * kp + 3
            wait_load(ib0, bf0, semL0)
            issue_scat(ib0, bf0, semS0)

            @pl.when(c1 < nch)
            def _():
                wait_load(ib1, bf1, semL1)
                issue_scat(ib1, bf1, semS1)

            @pl.when(c2 < nch)
            def _():
                wait_scat(ib0, bf0, semS0)
                issue_load(c2, ib0, bf0, semL0)

            @pl.when(c3 < nch)
            def _():
                wait_scat(ib1, bf1, semS1)
                issue_load(c3, ib1, bf1, semL1)

            return carry

        lax.fori_loop(0, (nch + 1) // 2, pairbody, 0)
        wait_scat(ib0, bf0, semS0)
        wait_scat(ib1, bf1, semS1)
        plsc.subcore_barrier()

        def outcopy(j, carry):
            off = pl.multiple_of((sid + j * NS) * zrows, 8)
            sl = pl.ds(off, zrows)
            pltpu.sync_copy(acc_sh.at[sl], s_hbm.at[cid, sl])
            return carry

        lax.fori_loop(0, nb, outcopy, 0)

    return k(msg, dst)


def _bn_affine(g, be, s, q):
    mu = s / N_EDGES
    var = q / N_EDGES - mu * mu
    a = g.reshape(1, -1) * lax.rsqrt(var + EPS)
    c = be.reshape(1, -1) - mu * a
    return a, c


def _pad_cols(m, w):
    return jnp.pad(m, ((0, 0), (0, w - m.shape[1])))


def _pad_rows(m, w):
    return jnp.pad(m, ((0, w - m.shape[0]), (0, 0)))


def _split_w1(p):
    w1 = p["W1"]
    d = w1.shape[1] // 2
    wa = w1[:, :d]
    wb = w1[:, d:]
    return (wa - wb).T, wb.T


def kernel(x, edge_index, edge_attr, params):
    del edge_attr
    src = edge_index[0].astype(jnp.int32)
    dst = edge_index[1].astype(jnp.int32)
    p0, p1, p2 = params["net0"], params["net1"], params["net2"]

    wd, wb = _split_w1(p0)
    u, v = _uv_from_x(x, _pad_cols(wd, 128), _pad_cols(wb, 128),
                      _pad_cols(p0["b1"].reshape(1, -1), 128).reshape(-1))
    h1 = _sc_gather(u, v, dst, src)
    s1, q1 = _stats(h1)
    a1, c1 = _bn_affine(p0["g1"], p0["be1"], s1[:, :64], q1[:, :64])
    h2, s2, q2 = _mlp_pass(h1, _pad_cols(a1, 128), _pad_cols(c1, 128),
                           _pad_rows(p0["W2"].T, 128), p0["b2"])
    a2, c2 = _bn_affine(p0["g2"], p0["be2"], s2, q2)
    h3, s3, q3 = _mlp_pass(h2, a2, c2, p0["W3"].T, p0["b3"])
    a3, c3 = _bn_affine(p0["g3"], p0["be3"], s3, q3)
    msg = _affine_pad(h3, a3, c3, 128, cnt_col=True)
    out0 = _sc_scatter(msg, dst, edge_split=True)

    wd, wb = _split_w1(p1)
    u, v = _uv_sum_halves(out0, out0, wd, wb, p1["b1"], 64)
    h1 = _sc_gather(u, v, dst, src)
    s1, q1 = _stats(h1)
    a1, c1 = _bn_affine(p1["g1"], p1["be1"], s1, q1)
    h2, s2, q2 = _mlp_pass(h1, a1, c1, p1["W2"].T, p1["b2"])
    a2, c2 = _bn_affine(p1["g2"], p1["be2"], s2, q2)
    h3, s3, q3 = _mlp_pass(h2, a2, c2, p1["W3"].T, p1["b3"])
    a3, c3 = _bn_affine(p1["g3"], p1["be3"], s3, q3)
    msg = _affine_pad(h3, a3, c3, 128, cnt_col=False)
    out1 = _sc_scatter(msg, dst, edge_split=True)

    wd, wb = _split_w1(p2)
    u, v = _uv_sum_halves(out1, out0, wd, wb, p2["b1"], 128)
    h1 = _sc_gather(u, v, dst, src)
    s1, q1 = _stats(h1)
    a1, c1 = _bn_affine(p2["g1"], p2["be1"], s1, q1)
    h2, s2, q2 = _mlp_pass(h1, a1, c1, p2["W2"].T, p2["b2"])
    a2, c2 = _bn_affine(p2["g2"], p2["be2"], s2, q2)
    h3, s3, q3 = _mlp_pass(h2, a2, c2, p2["W3"].T, p2["b3"])
    a3, c3 = _bn_affine(p2["g3"], p2["be3"], s3, q3)
    msg2 = _affine_split(h3, a3, c3)
    out2 = _sc_scatter(msg2, dst, edge_split=False)

    return _head(out2, out0, params["lin"]["W"].T, params["lin"]["b"])


# speedup vs baseline: 1.0430x; 1.0430x over previous
import functools

import jax
import jax.numpy as jnp
from jax import lax
from jax.experimental import pallas as pl
from jax.experimental.pallas import tpu as pltpu
from jax.experimental.pallas import tpu_sc as plsc

N_NODES = 10000
N_EDGES = 320000
EPS = 1e-5

NC = 2
NS = 16
NW = NC * NS
CHUNK = 80
CNT_COL = 64

f32 = jnp.float32

_SC_MESH = dict(core_axis_name="c", subcore_axis_name="s")


_R_EDGE = 2560
_R_NODE = 2000

_ARB = pltpu.CompilerParams(dimension_semantics=("arbitrary",))


def _uv_from_x(x, Wd, Wb, b1):
    n, d = x.shape
    h = Wd.shape[1]

    def kern(x_ref, wd_ref, wb_ref, b1_ref, u_ref, v_ref):
        y = x_ref[...]
        u_ref[...] = jnp.dot(y, wd_ref[...], preferred_element_type=f32) + b1_ref[...]
        v_ref[...] = jnp.dot(y, wb_ref[...], preferred_element_type=f32)

    return pl.pallas_call(
        kern,
        grid=(n // _R_NODE,),
        in_specs=[
            pl.BlockSpec((_R_NODE, d), lambda i: (i, 0)),
            pl.BlockSpec((d, h), lambda i: (0, 0)),
            pl.BlockSpec((d, h), lambda i: (0, 0)),
            pl.BlockSpec((1, h), lambda i: (0, 0)),
        ],
        out_specs=[
            pl.BlockSpec((_R_NODE, h), lambda i: (i, 0)),
            pl.BlockSpec((_R_NODE, h), lambda i: (i, 0)),
        ],
        out_shape=[
            jax.ShapeDtypeStruct((n, h), f32),
            jax.ShapeDtypeStruct((n, h), f32),
        ],
        compiler_params=_ARB,
    )(x, Wd, Wb, b1.reshape(1, h))


def _uv_sum_halves(s_arr, cnt_arr, Wd, Wb, b1, d_use):
    _, n, w = s_arr.shape
    h = Wd.shape[1]

    def kern(s_ref, cnt_ref, wd_ref, wb_ref, b1_ref, u_ref, v_ref):
        cnt = (cnt_ref[0, :, CNT_COL:CNT_COL + 1]
               + cnt_ref[1, :, CNT_COL:CNT_COL + 1])
        inv = 1.0 / jnp.maximum(cnt, 1.0)
        tot = s_ref[0] + s_ref[1]
        y = tot[:, :d_use] * inv
        u_ref[...] = jnp.dot(y, wd_ref[...], preferred_element_type=f32) + b1_ref[...]
        v_ref[...] = jnp.dot(y, wb_ref[...], preferred_element_type=f32)

    return pl.pallas_call(
        kern,
        grid=(n // _R_NODE,),
        in_specs=[
            pl.BlockSpec((2, _R_NODE, w), lambda i: (0, i, 0)),
            pl.BlockSpec((2, _R_NODE, 128), lambda i: (0, i, 0)),
            pl.BlockSpec((d_use, h), lambda i: (0, 0)),
            pl.BlockSpec((d_use, h), lambda i: (0, 0)),
            pl.BlockSpec((1, h), lambda i: (0, 0)),
        ],
        out_specs=[
            pl.BlockSpec((_R_NODE, h), lambda i: (i, 0)),
            pl.BlockSpec((_R_NODE, h), lambda i: (i, 0)),
        ],
        out_shape=[
            jax.ShapeDtypeStruct((n, h), f32),
            jax.ShapeDtypeStruct((n, h), f32),
        ],
        compiler_params=_ARB,
    )(s_arr, cnt_arr, Wd, Wb, b1.reshape(1, h))


def _stats(h1):
    e, h = h1.shape

    def kern(h_ref, s_ref, q_ref):
        @pl.when(pl.program_id(0) == 0)
        def _():
            s_ref[...] = jnp.zeros_like(s_ref)
            q_ref[...] = jnp.zeros_like(q_ref)

        v = h_ref[...]
        s_ref[...] += jnp.sum(v, axis=0, keepdims=True)
        q_ref[...] += jnp.sum(v * v, axis=0, keepdims=True)

    return pl.pallas_call(
        kern,
        grid=(e // _R_EDGE,),
        in_specs=[pl.BlockSpec((_R_EDGE, h), lambda i: (i, 0))],
        out_specs=[
            pl.BlockSpec((1, h), lambda i: (0, 0)),
            pl.BlockSpec((1, h), lambda i: (0, 0)),
        ],
        out_shape=[
            jax.ShapeDtypeStruct((1, h), f32),
            jax.ShapeDtypeStruct((1, h), f32),
        ],
        compiler_params=_ARB,
    )(h1)


def _bdot(m, w):
    return jnp.dot(m.astype(jnp.bfloat16), w.astype(jnp.bfloat16),
                   preferred_element_type=f32)


def _mlp_stats(h_in, a, c, W, b):
    e, hi = h_in.shape
    ho = W.shape[1]

    def kern(h_ref, a_ref, c_ref, w_ref, b_ref, s_ref, q_ref):
        @pl.when(pl.program_id(0) == 0)
        def _():
            s_ref[...] = jnp.zeros_like(s_ref)
            q_ref[...] = jnp.zeros_like(q_ref)

        m = jnp.maximum(h_ref[...] * a_ref[...] + c_ref[...], 0.0)
        v = _bdot(m, w_ref[...]) + b_ref[...]
        s_ref[...] += jnp.sum(v, axis=0, keepdims=True)
        q_ref[...] += jnp.sum(v * v, axis=0, keepdims=True)

    return pl.pallas_call(
        kern,
        grid=(e // _R_EDGE,),
        in_specs=[
            pl.BlockSpec((_R_EDGE, hi), lambda i: (i, 0)),
            pl.BlockSpec((1, hi), lambda i: (0, 0)),
            pl.BlockSpec((1, hi), lambda i: (0, 0)),
            pl.BlockSpec((hi, ho), lambda i: (0, 0)),
            pl.BlockSpec((1, ho), lambda i: (0, 0)),
        ],
        out_specs=[
            pl.BlockSpec((1, ho), lambda i: (0, 0)),
            pl.BlockSpec((1, ho), lambda i: (0, 0)),
        ],
        out_shape=[
            jax.ShapeDtypeStruct((1, ho), f32),
            jax.ShapeDtypeStruct((1, ho), f32),
        ],
        compiler_params=_ARB,
    )(h_in, a, c, W, b.reshape(1, ho))


def _mlp2_stats(h_in, a1, c1, W2, b2, a2, c2, W3, b3):
    e, hi = h_in.shape
    ho = W3.shape[1]

    def kern(h_ref, a1_ref, c1_ref, w2_ref, b2_ref,
             a2_ref, c2_ref, w3_ref, b3_ref, s_ref, q_ref):
        @pl.when(pl.program_id(0) == 0)
        def _():
            s_ref[...] = jnp.zeros_like(s_ref)
            q_ref[...] = jnp.zeros_like(q_ref)

        m1 = jnp.maximum(h_ref[...] * a1_ref[...] + c1_ref[...], 0.0)
        h2 = _bdot(m1, w2_ref[...]) + b2_ref[...]
        m2 = jnp.maximum(h2 * a2_ref[...] + c2_ref[...], 0.0)
        v = _bdot(m2, w3_ref[...]) + b3_ref[...]
        s_ref[...] += jnp.sum(v, axis=0, keepdims=True)
        q_ref[...] += jnp.sum(v * v, axis=0, keepdims=True)

    hm = W2.shape[1]
    return pl.pallas_call(
        kern,
        grid=(e // _R_EDGE,),
        in_specs=[
            pl.BlockSpec((_R_EDGE, hi), lambda i: (i, 0)),
            pl.BlockSpec((1, hi), lambda i: (0, 0)),
            pl.BlockSpec((1, hi), lambda i: (0, 0)),
            pl.BlockSpec((hi, hm), lambda i: (0, 0)),
            pl.BlockSpec((1, hm), lambda i: (0, 0)),
            pl.BlockSpec((1, hm), lambda i: (0, 0)),
            pl.BlockSpec((1, hm), lambda i: (0, 0)),
            pl.BlockSpec((hm, ho), lambda i: (0, 0)),
            pl.BlockSpec((1, ho), lambda i: (0, 0)),
        ],
        out_specs=[
            pl.BlockSpec((1, ho), lambda i: (0, 0)),
            pl.BlockSpec((1, ho), lambda i: (0, 0)),
        ],
        out_shape=[
            jax.ShapeDtypeStruct((1, ho), f32),
            jax.ShapeDtypeStruct((1, ho), f32),
        ],
        compiler_params=_ARB,
    )(h_in, a1, c1, W2, b2.reshape(1, hm), a2, c2, W3, b3.reshape(1, ho))


def _mlp2_msg(h_in, a1, c1, W2, b2, a2, c2, W3, b3, a3, c3, wout, cnt_col, split):
    e, hi = h_in.shape
    ho = W3.shape[1]
    hm = W2.shape[1]
    extra = 0 if split else wout - ho
    h2o = ho // 2

    def kern(h_ref, a1_ref, c1_ref, w2_ref, b2_ref,
             a2_ref, c2_ref, w3_ref, b3_ref, a3_ref, c3_ref, o_ref):
        m1 = jnp.maximum(h_ref[...] * a1_ref[...] + c1_ref[...], 0.0)
        h2 = _bdot(m1, w2_ref[...]) + b2_ref[...]
        m2 = jnp.maximum(h2 * a2_ref[...] + c2_ref[...], 0.0)
        h3 = _bdot(m2, w3_ref[...]) + b3_ref[...]
        m = jnp.maximum(h3 * a3_ref[...] + c3_ref[...], 0.0)
        if split:
            o_ref[0] = m[:, :h2o]
            o_ref[1] = m[:, h2o:]
        elif extra == 0:
            o_ref[...] = m
        else:
            if cnt_col:
                col = lax.broadcasted_iota(jnp.int32, (_R_EDGE, extra), 1)
                pad = jnp.where(col == 0, 1.0, 0.0).astype(f32)
            else:
                pad = jnp.zeros((_R_EDGE, extra), f32)
            o_ref[...] = jnp.concatenate([m, pad], axis=-1)

    if split:
        out_spec = [pl.BlockSpec((2, _R_EDGE, h2o), lambda i: (0, i, 0))]
        out_shape = [jax.ShapeDtypeStruct((2, e, h2o), f32)]
    else:
        out_spec = [pl.BlockSpec((_R_EDGE, wout), lambda i: (i, 0))]
        out_shape = [jax.ShapeDtypeStruct((e, wout), f32)]

    return pl.pallas_call(
        kern,
        grid=(e // _R_EDGE,),
        in_specs=[
            pl.BlockSpec((_R_EDGE, hi), lambda i: (i, 0)),
            pl.BlockSpec((1, hi), lambda i: (0, 0)),
            pl.BlockSpec((1, hi), lambda i: (0, 0)),
            pl.BlockSpec((hi, hm), lambda i: (0, 0)),
            pl.BlockSpec((1, hm), lambda i: (0, 0)),
            pl.BlockSpec((1, hm), lambda i: (0, 0)),
            pl.BlockSpec((1, hm), lambda i: (0, 0)),
            pl.BlockSpec((hm, ho), lambda i: (0, 0)),
            pl.BlockSpec((1, ho), lambda i: (0, 0)),
            pl.BlockSpec((1, ho), lambda i: (0, 0)),
            pl.BlockSpec((1, ho), lambda i: (0, 0)),
        ],
        out_specs=out_spec,
        out_shape=out_shape,
        compiler_params=_ARB,
    )(h_in, a1, c1, W2, b2.reshape(1, hm), a2, c2,
      W3, b3.reshape(1, ho), a3, c3)[0]


def _head(s_arr, cnt_arr, Wl, bl):
    _, n, hp2 = s_arr.shape
    d = 2 * hp2

    def kern(s_ref, cnt_ref, w_ref, b_ref, o_ref):
        cnt = (cnt_ref[0, :, CNT_COL:CNT_COL + 1]
               + cnt_ref[1, :, CNT_COL:CNT_COL + 1])
        inv = 1.0 / jnp.maximum(cnt, 1.0)
        y = jnp.concatenate([s_ref[0], s_ref[1]], axis=-1) * inv
        z = jnp.dot(y, w_ref[...], preferred_element_type=f32) + b_ref[...]
        o_ref[...] = jax.nn.sigmoid(z)

    return pl.pallas_call(
        kern,
        grid=(n // _R_NODE,),
        in_specs=[
            pl.BlockSpec((2, _R_NODE, hp2), lambda i: (0, i, 0)),
            pl.BlockSpec((2, _R_NODE, 128), lambda i: (0, i, 0)),
            pl.BlockSpec((d, 1), lambda i: (0, 0)),
            pl.BlockSpec((1, 1), lambda i: (0, 0)),
        ],
        out_specs=[pl.BlockSpec((_R_NODE, 1), lambda i: (i, 0))],
        out_shape=[jax.ShapeDtypeStruct((n, 1), f32)],
        compiler_params=_ARB,
    )(s_arr, cnt_arr, Wl, bl.reshape(1, 1))[0]


def _sc_gather(U, V, dst, src):
    n, h = U.shape
    hf = h // 16
    ept = N_EDGES // NW
    nch = ept // CHUNK

    mesh = plsc.VectorSubcoreMesh(**_SC_MESH)

    @functools.partial(
        pl.kernel,
        mesh=mesh,
        out_type=jax.ShapeDtypeStruct((N_EDGES, h), f32),
        scratch_types=[
            pltpu.VMEM((CHUNK,), jnp.int32),
            pltpu.VMEM((CHUNK,), jnp.int32),
            pltpu.VMEM((CHUNK,), jnp.int32),
            pltpu.VMEM((CHUNK,), jnp.int32),
            pltpu.VMEM((CHUNK, h), f32),
            pltpu.VMEM((CHUNK, h), f32),
            pltpu.VMEM((CHUNK, h), f32),
            pltpu.VMEM((CHUNK, h), f32),
            pltpu.SemaphoreType.DMA,
            pltpu.SemaphoreType.DMA,
            pltpu.SemaphoreType.DMA,
            pltpu.SemaphoreType.DMA,
            pltpu.SemaphoreType.DMA,
            pltpu.SemaphoreType.DMA,
        ],
    )
    def k(u_hbm, v_hbm, dst_hbm, src_hbm, h1_hbm,
          id0, is0, id1, is1, ra0, rb0, ra1, rb1,
          semi0, semi1, semr0, semr1, semw0, semw1):
        cid = lax.axis_index("c")
        sid = lax.axis_index("s")
        wid = sid * NC + cid
        base = wid * ept

        def off(c):
            return pl.multiple_of(base + c * CHUNK, 8)

        def issue_idx(c, id_, is_, sem):
            pltpu.async_copy(dst_hbm.at[pl.ds(off(c), CHUNK)], id_, sem)
            pltpu.async_copy(src_hbm.at[pl.ds(off(c), CHUNK)], is_, sem)

        def wait_idx(id_, is_, sem):
            pltpu.make_async_copy(dst_hbm.at[pl.ds(0, CHUNK)], id_, sem).wait()
            pltpu.make_async_copy(src_hbm.at[pl.ds(0, CHUNK)], is_, sem).wait()

        def issue_gather(id_, is_, ra_, rb_, sem):
            pltpu.async_copy(u_hbm.at[id_], ra_, sem)
            pltpu.async_copy(v_hbm.at[is_], rb_, sem)

        def wait_gather(id_, is_, ra_, rb_, sem):
            pltpu.make_async_copy(u_hbm.at[id_], ra_, sem).wait()
            pltpu.make_async_copy(v_hbm.at[is_], rb_, sem).wait()

        def compute(ra_, rb_):
            def rowfn(r, carry2):
                for f in range(hf):
                    sl = pl.ds(f * 16, 16)
                    plsc.addupdate(ra_.at[r, sl], rb_[r, sl])
                return carry2

            lax.fori_loop(0, CHUNK, rowfn, 0)

        def issue_write(c, ra_, sem):
            pltpu.async_copy(ra_, h1_hbm.at[pl.ds(off(c), CHUNK)], sem)

        def wait_write(ra_, sem):
            pltpu.make_async_copy(ra_, h1_hbm.at[pl.ds(0, CHUNK)], sem).wait()

        issue_idx(0, id0, is0, semi0)
        wait_idx(id0, is0, semi0)
        issue_gather(id0, is0, ra0, rb0, semr0)
        issue_idx(1, id1, is1, semi1)

        def pairbody(kp, carry):
            c1 = 2 * kp + 1
            c2 = 2 * kp + 2
            c3 = 2 * kp + 3
            wait_gather(id0, is0, ra0, rb0, semr0)
            compute(ra0, rb0)
            issue_write(2 * kp, ra0, semw0)

            @pl.when(c1 < nch)
            def _():
                wait_idx(id1, is1, semi1)

                @pl.when(kp > 0)
                def _():
                    wait_write(ra1, semw1)

                issue_gather(id1, is1, ra1, rb1, semr1)

                @pl.when(c2 < nch)
                def _():
                    issue_idx(c2, id0, is0, semi0)

            @pl.when(c1 < nch)
            def _():
                wait_gather(id1, is1, ra1, rb1, semr1)
                compute(ra1, rb1)
                issue_write(c1, ra1, semw1)

                @pl.when(c2 < nch)
                def _():
                    wait_idx(id0, is0, semi0)
                    wait_write(ra0, semw0)
                    issue_gather(id0, is0, ra0, rb0, semr0)

                    @pl.when(c3 < nch)
                    def _():
                        issue_idx(c3, id1, is1, semi1)

            return carry

        lax.fori_loop(0, (nch + 1) // 2, pairbody, 0)
        wait_write(ra0, semw0)
        wait_write(ra1, semw1)

    return k(U, V, dst, src)


def _sc_scatter(msg, dst, edge_split):
    if edge_split:
        ept = N_EDGES // 2 // NS
    else:
        ept = N_EDGES // NS
    nch = ept // CHUNK
    w = 128
    zrows = 80
    nblk = N_NODES // zrows

    mesh = plsc.VectorSubcoreMesh(**_SC_MESH)

    @functools.partial(
        pl.kernel,
        mesh=mesh,
        out_type=jax.ShapeDtypeStruct((NC, N_NODES, w), f32),
        scratch_types=[
            pltpu.VMEM((CHUNK,), jnp.int32),
            pltpu.VMEM((CHUNK,), jnp.int32),
            pltpu.VMEM((CHUNK, w), f32),
            pltpu.VMEM((CHUNK, w), f32),
            pltpu.VMEM((zrows, w), f32),
            pltpu.VMEM_SHARED((N_NODES, w), f32),
            pltpu.SemaphoreType.DMA,
            pltpu.SemaphoreType.DMA,
            pltpu.SemaphoreType.DMA,
            pltpu.SemaphoreType.DMA,
        ],
    )
    def k(msg_hbm, dst_hbm, s_hbm, ib0, ib1, bf0, bf1, zbuf, acc_sh,
          semL0, semL1, semS0, semS1):
        cid = lax.axis_index("c")
        sid = lax.axis_index("s")
        zero = jnp.zeros((16,), f32)

        def zf(r, carry):
            for f in range(w // 16):
                zbuf[r, pl.ds(f * 16, 16)] = zero
            return carry

        lax.fori_loop(0, zrows, zf, 0)
        nb = jnp.where(sid < nblk - (nblk // NS) * NS, nblk // NS + 1, nblk // NS)

        def zcopy(j, carry):
            off = pl.multiple_of((sid + j * NS) * zrows, 8)
            pltpu.sync_copy(zbuf, acc_sh.at[pl.ds(off, zrows)])
            return carry

        lax.fori_loop(0, nb, zcopy, 0)
        plsc.subcore_barrier()

        if edge_split:
            base = cid * (N_EDGES // 2) + sid * ept
        else:
            base = sid * ept

        def off(c):
            return pl.multiple_of(base + c * CHUNK, 8)

        def msg_slice(c):
            if edge_split:
                return msg_hbm.at[pl.ds(off(c), CHUNK)]
            return msg_hbm.at[cid, pl.ds(off(c), CHUNK)]

        def issue_load(c, ib, bf, sem):
            pltpu.async_copy(dst_hbm.at[pl.ds(off(c), CHUNK)], ib, sem)
            pltpu.async_copy(msg_slice(c), bf, sem)

        def wait_load(ib, bf, sem):
            pltpu.make_async_copy(dst_hbm.at[pl.ds(0, CHUNK)], ib, sem).wait()
            pltpu.make_async_copy(msg_slice(0), bf, sem).wait()

        def issue_scat(ib, bf, sem):
            pltpu.async_copy(bf, acc_sh.at[ib], sem, add=True)

        def wait_scat(ib, bf, sem):
            pltpu.make_async_copy(bf, acc_sh.at[ib], sem).wait()

        issue_load(0, ib0, bf0, semL0)
        issue_load(1, ib1, bf1, semL1)

        def pairbody(kp, carry):
            c1 = 2 * kp + 1
            c2 = 2 * kp + 2
            c3 = 2 * kp + 3
            wait_load(ib0, bf0, semL0)
            issue_scat(ib0, bf0, semS0)

            @pl.when(c1 < nch)
            def _():
                wait_load(ib1, bf1, semL1)
                issue_scat(ib1, bf1, semS1)

            @pl.when(c2 < nch)
            def _():
                wait_scat(ib0, bf0, semS0)
                issue_load(c2, ib0, bf0, semL0)

            @pl.when(c3 < nch)
            def _():
                wait_scat(ib1, bf1, semS1)
                issue_load(c3, ib1, bf1, semL1)

            return carry

        lax.fori_loop(0, (nch + 1) // 2, pairbody, 0)
        wait_scat(ib0, bf0, semS0)
        wait_scat(ib1, bf1, semS1)
        plsc.subcore_barrier()

        def outcopy(j, carry):
            off = pl.multiple_of((sid + j * NS) * zrows, 8)
            sl = pl.ds(off, zrows)
            pltpu.sync_copy(acc_sh.at[sl], s_hbm.at[cid, sl])
            return carry

        lax.fori_loop(0, nb, outcopy, 0)

    return k(msg, dst)


def _bn_affine(g, be, s, q):
    mu = s / N_EDGES
    var = q / N_EDGES - mu * mu
    a = g.reshape(1, -1) * lax.rsqrt(var + EPS)
    c = be.reshape(1, -1) - mu * a
    return a, c


def _pad_cols(m, w):
    return jnp.pad(m, ((0, 0), (0, w - m.shape[1])))


def _pad_rows(m, w):
    return jnp.pad(m, ((0, w - m.shape[0]), (0, 0)))


def _split_w1(p):
    w1 = p["W1"]
    d = w1.shape[1] // 2
    wa = w1[:, :d]
    wb = w1[:, d:]
    return (wa - wb).T, wb.T


def _layer_passes(h1, a1, c1, p, w2, wout, cnt_col, split):
    w3 = p["W3"].T
    s2, q2 = _mlp_stats(h1, a1, c1, w2, p["b2"])
    a2, c2 = _bn_affine(p["g2"], p["be2"], s2, q2)
    s3, q3 = _mlp2_stats(h1, a1, c1, w2, p["b2"], a2, c2, w3, p["b3"])
    a3, c3 = _bn_affine(p["g3"], p["be3"], s3, q3)
    return _mlp2_msg(h1, a1, c1, w2, p["b2"], a2, c2, w3, p["b3"],
                     a3, c3, wout, cnt_col, split)


def kernel(x, edge_index, edge_attr, params):
    del edge_attr
    src = edge_index[0].astype(jnp.int32)
    dst = edge_index[1].astype(jnp.int32)
    p0, p1, p2 = params["net0"], params["net1"], params["net2"]

    wd, wb = _split_w1(p0)
    u, v = _uv_from_x(x, _pad_cols(wd, 128), _pad_cols(wb, 128),
                      _pad_cols(p0["b1"].reshape(1, -1), 128).reshape(-1))
    h1 = _sc_gather(u, v, dst, src)
    s1, q1 = _stats(h1)
    a1, c1 = _bn_affine(p0["g1"], p0["be1"], s1[:, :64], q1[:, :64])
    msg = _layer_passes(h1, _pad_cols(a1, 128), _pad_cols(c1, 128), p0,
                        _pad_rows(p0["W2"].T, 128), 128,
                        cnt_col=True, split=False)
    out0 = _sc_scatter(msg, dst, edge_split=True)

    wd, wb = _split_w1(p1)
    u, v = _uv_sum_halves(out0, out0, wd, wb, p1["b1"], 64)
    h1 = _sc_gather(u, v, dst, src)
    s1, q1 = _stats(h1)
    a1, c1 = _bn_affine(p1["g1"], p1["be1"], s1, q1)
    msg = _layer_passes(h1, a1, c1, p1, p1["W2"].T, 128,
                        cnt_col=False, split=False)
    out1 = _sc_scatter(msg, dst, edge_split=True)

    wd, wb = _split_w1(p2)
    u, v = _uv_sum_halves(out1, out0, wd, wb, p2["b1"], 128)
    h1 = _sc_gather(u, v, dst, src)
    s1, q1 = _stats(h1)
    a1, c1 = _bn_affine(p2["g1"], p2["be1"], s1, q1)
    msg2 = _layer_passes(h1, a1, c1, p2, p2["W2"].T, 0,
                         cnt_col=False, split=True)
    out2 = _sc_scatter(msg2, dst, edge_split=False)

    return _head(out2, out0, params["lin"]["W"].T, params["lin"]["b"])


# speedup vs baseline: 1.0458x; 1.0027x over previous
import functools

import jax
import jax.numpy as jnp
from jax import lax
from jax.experimental import pallas as pl
from jax.experimental.pallas import tpu as pltpu
from jax.experimental.pallas import tpu_sc as plsc

N_NODES = 10000
N_EDGES = 320000
EPS = 1e-5

NC = 2
NS = 16
NW = NC * NS
CHUNK = 80
CNT_COL = 64

f32 = jnp.float32

_SC_MESH = dict(core_axis_name="c", subcore_axis_name="s")


_R_EDGE = 2560
_R_NODE = 2000

_ARB = pltpu.CompilerParams(dimension_semantics=("arbitrary",))


def _uv_from_x(x, Wd, Wb, b1):
    n, d = x.shape
    h = Wd.shape[1]

    def kern(x_ref, wd_ref, wb_ref, b1_ref, u_ref, v_ref):
        y = x_ref[...]
        u_ref[...] = jnp.dot(y, wd_ref[...], preferred_element_type=f32) + b1_ref[...]
        v_ref[...] = jnp.dot(y, wb_ref[...], preferred_element_type=f32)

    return pl.pallas_call(
        kern,
        grid=(n // _R_NODE,),
        in_specs=[
            pl.BlockSpec((_R_NODE, d), lambda i: (i, 0)),
            pl.BlockSpec((d, h), lambda i: (0, 0)),
            pl.BlockSpec((d, h), lambda i: (0, 0)),
            pl.BlockSpec((1, h), lambda i: (0, 0)),
        ],
        out_specs=[
            pl.BlockSpec((_R_NODE, h), lambda i: (i, 0)),
            pl.BlockSpec((_R_NODE, h), lambda i: (i, 0)),
        ],
        out_shape=[
            jax.ShapeDtypeStruct((n, h), f32),
            jax.ShapeDtypeStruct((n, h), f32),
        ],
        compiler_params=_ARB,
    )(x, Wd, Wb, b1.reshape(1, h))


def _uv_sum_halves(s_arr, cnt_arr, Wd, Wb, b1, d_use):
    _, n, w = s_arr.shape
    h = Wd.shape[1]

    def kern(s_ref, cnt_ref, wd_ref, wb_ref, b1_ref, u_ref, v_ref):
        cnt = (cnt_ref[0, :, CNT_COL:CNT_COL + 1]
               + cnt_ref[1, :, CNT_COL:CNT_COL + 1])
        inv = 1.0 / jnp.maximum(cnt, 1.0)
        tot = s_ref[0] + s_ref[1]
        y = tot[:, :d_use] * inv
        u_ref[...] = jnp.dot(y, wd_ref[...], preferred_element_type=f32) + b1_ref[...]
        v_ref[...] = jnp.dot(y, wb_ref[...], preferred_element_type=f32)

    return pl.pallas_call(
        kern,
        grid=(n // _R_NODE,),
        in_specs=[
            pl.BlockSpec((2, _R_NODE, w), lambda i: (0, i, 0)),
            pl.BlockSpec((2, _R_NODE, 128), lambda i: (0, i, 0)),
            pl.BlockSpec((d_use, h), lambda i: (0, 0)),
            pl.BlockSpec((d_use, h), lambda i: (0, 0)),
            pl.BlockSpec((1, h), lambda i: (0, 0)),
        ],
        out_specs=[
            pl.BlockSpec((_R_NODE, h), lambda i: (i, 0)),
            pl.BlockSpec((_R_NODE, h), lambda i: (i, 0)),
        ],
        out_shape=[
            jax.ShapeDtypeStruct((n, h), f32),
            jax.ShapeDtypeStruct((n, h), f32),
        ],
        compiler_params=_ARB,
    )(s_arr, cnt_arr, Wd, Wb, b1.reshape(1, h))


def _stats(h1):
    e, h = h1.shape

    def kern(h_ref, s_ref, q_ref):
        @pl.when(pl.program_id(0) == 0)
        def _():
            s_ref[...] = jnp.zeros_like(s_ref)
            q_ref[...] = jnp.zeros_like(q_ref)

        v = h_ref[...]
        s_ref[...] += jnp.sum(v, axis=0, keepdims=True)
        q_ref[...] += jnp.sum(v * v, axis=0, keepdims=True)

    return pl.pallas_call(
        kern,
        grid=(e // _R_EDGE,),
        in_specs=[pl.BlockSpec((_R_EDGE, h), lambda i: (i, 0))],
        out_specs=[
            pl.BlockSpec((1, h), lambda i: (0, 0)),
            pl.BlockSpec((1, h), lambda i: (0, 0)),
        ],
        out_shape=[
            jax.ShapeDtypeStruct((1, h), f32),
            jax.ShapeDtypeStruct((1, h), f32),
        ],
        compiler_params=_ARB,
    )(h1)


def _bdot(m, w):
    return jnp.dot(m, w, preferred_element_type=f32)


def _mlp_stats(h_in, a, c, W, b):
    e, hi = h_in.shape
    ho = W.shape[1]

    def kern(h_ref, a_ref, c_ref, w_ref, b_ref, s_ref, q_ref):
        @pl.when(pl.program_id(0) == 0)
        def _():
            s_ref[...] = jnp.zeros_like(s_ref)
            q_ref[...] = jnp.zeros_like(q_ref)

        m = jnp.maximum(h_ref[...] * a_ref[...] + c_ref[...], 0.0)
        v = _bdot(m, w_ref[...]) + b_ref[...]
        s_ref[...] += jnp.sum(v, axis=0, keepdims=True)
        q_ref[...] += jnp.sum(v * v, axis=0, keepdims=True)

    return pl.pallas_call(
        kern,
        grid=(e // _R_EDGE,),
        in_specs=[
            pl.BlockSpec((_R_EDGE, hi), lambda i: (i, 0)),
            pl.BlockSpec((1, hi), lambda i: (0, 0)),
            pl.BlockSpec((1, hi), lambda i: (0, 0)),
            pl.BlockSpec((hi, ho), lambda i: (0, 0)),
            pl.BlockSpec((1, ho), lambda i: (0, 0)),
        ],
        out_specs=[
            pl.BlockSpec((1, ho), lambda i: (0, 0)),
            pl.BlockSpec((1, ho), lambda i: (0, 0)),
        ],
        out_shape=[
            jax.ShapeDtypeStruct((1, ho), f32),
            jax.ShapeDtypeStruct((1, ho), f32),
        ],
        compiler_params=_ARB,
    )(h_in, a, c, W, b.reshape(1, ho))


def _mlp2_stats(h_in, a1, c1, W2, b2, a2, c2, W3, b3):
    e, hi = h_in.shape
    ho = W3.shape[1]

    def kern(h_ref, a1_ref, c1_ref, w2_ref, b2_ref,
             a2_ref, c2_ref, w3_ref, b3_ref, s_ref, q_ref):
        @pl.when(pl.program_id(0) == 0)
        def _():
            s_ref[...] = jnp.zeros_like(s_ref)
            q_ref[...] = jnp.zeros_like(q_ref)

        m1 = jnp.maximum(h_ref[...] * a1_ref[...] + c1_ref[...], 0.0)
        h2 = _bdot(m1, w2_ref[...]) + b2_ref[...]
        m2 = jnp.maximum(h2 * a2_ref[...] + c2_ref[...], 0.0)
        v = _bdot(m2, w3_ref[...]) + b3_ref[...]
        s_ref[...] += jnp.sum(v, axis=0, keepdims=True)
        q_ref[...] += jnp.sum(v * v, axis=0, keepdims=True)

    hm = W2.shape[1]
    return pl.pallas_call(
        kern,
        grid=(e // _R_EDGE,),
        in_specs=[
            pl.BlockSpec((_R_EDGE, hi), lambda i: (i, 0)),
            pl.BlockSpec((1, hi), lambda i: (0, 0)),
            pl.BlockSpec((1, hi), lambda i: (0, 0)),
            pl.BlockSpec((hi, hm), lambda i: (0, 0)),
            pl.BlockSpec((1, hm), lambda i: (0, 0)),
            pl.BlockSpec((1, hm), lambda i: (0, 0)),
            pl.BlockSpec((1, hm), lambda i: (0, 0)),
            pl.BlockSpec((hm, ho), lambda i: (0, 0)),
            pl.BlockSpec((1, ho), lambda i: (0, 0)),
        ],
        out_specs=[
            pl.BlockSpec((1, ho), lambda i: (0, 0)),
            pl.BlockSpec((1, ho), lambda i: (0, 0)),
        ],
        out_shape=[
            jax.ShapeDtypeStruct((1, ho), f32),
            jax.ShapeDtypeStruct((1, ho), f32),
        ],
        compiler_params=_ARB,
    )(h_in, a1, c1, W2, b2.reshape(1, hm), a2, c2, W3, b3.reshape(1, ho))


def _mlp2_msg(h_in, a1, c1, W2, b2, a2, c2, W3, b3, a3, c3, wout, cnt_col, split):
    e, hi = h_in.shape
    ho = W3.shape[1]
    hm = W2.shape[1]
    extra = 0 if split else wout - ho
    h2o = ho // 2

    def kern(h_ref, a1_ref, c1_ref, w2_ref, b2_ref,
             a2_ref, c2_ref, w3_ref, b3_ref, a3_ref, c3_ref, o_ref):
        m1 = jnp.maximum(h_ref[...] * a1_ref[...] + c1_ref[...], 0.0)
        h2 = _bdot(m1, w2_ref[...]) + b2_ref[...]
        m2 = jnp.maximum(h2 * a2_ref[...] + c2_ref[...], 0.0)
        h3 = _bdot(m2, w3_ref[...]) + b3_ref[...]
        m = jnp.maximum(h3 * a3_ref[...] + c3_ref[...], 0.0)
        if split:
            o_ref[0] = m[:, :h2o]
            o_ref[1] = m[:, h2o:]
        elif extra == 0:
            o_ref[...] = m
        else:
            if cnt_col:
                col = lax.broadcasted_iota(jnp.int32, (_R_EDGE, extra), 1)
                pad = jnp.where(col == 0, 1.0, 0.0).astype(f32)
            else:
                pad = jnp.zeros((_R_EDGE, extra), f32)
            o_ref[...] = jnp.concatenate([m, pad], axis=-1)

    if split:
        out_spec = [pl.BlockSpec((2, _R_EDGE, h2o), lambda i: (0, i, 0))]
        out_shape = [jax.ShapeDtypeStruct((2, e, h2o), f32)]
    else:
        out_spec = [pl.BlockSpec((_R_EDGE, wout), lambda i: (i, 0))]
        out_shape = [jax.ShapeDtypeStruct((e, wout), f32)]

    return pl.pallas_call(
        kern,
        grid=(e // _R_EDGE,),
        in_specs=[
            pl.BlockSpec((_R_EDGE, hi), lambda i: (i, 0)),
            pl.BlockSpec((1, hi), lambda i: (0, 0)),
            pl.BlockSpec((1, hi), lambda i: (0, 0)),
            pl.BlockSpec((hi, hm), lambda i: (0, 0)),
            pl.BlockSpec((1, hm), lambda i: (0, 0)),
            pl.BlockSpec((1, hm), lambda i: (0, 0)),
            pl.BlockSpec((1, hm), lambda i: (0, 0)),
            pl.BlockSpec((hm, ho), lambda i: (0, 0)),
            pl.BlockSpec((1, ho), lambda i: (0, 0)),
            pl.BlockSpec((1, ho), lambda i: (0, 0)),
            pl.BlockSpec((1, ho), lambda i: (0, 0)),
        ],
        out_specs=out_spec,
        out_shape=out_shape,
        compiler_params=_ARB,
    )(h_in, a1, c1, W2, b2.reshape(1, hm), a2, c2,
      W3, b3.reshape(1, ho), a3, c3)[0]


def _head(s_arr, cnt_arr, Wl, bl):
    _, n, hp2 = s_arr.shape
    d = 2 * hp2

    def kern(s_ref, cnt_ref, w_ref, b_ref, o_ref):
        cnt = (cnt_ref[0, :, CNT_COL:CNT_COL + 1]
               + cnt_ref[1, :, CNT_COL:CNT_COL + 1])
        inv = 1.0 / jnp.maximum(cnt, 1.0)
        y = jnp.concatenate([s_ref[0], s_ref[1]], axis=-1) * inv
        z = jnp.dot(y, w_ref[...], preferred_element_type=f32) + b_ref[...]
        o_ref[...] = jax.nn.sigmoid(z)

    return pl.pallas_call(
        kern,
        grid=(n // _R_NODE,),
        in_specs=[
            pl.BlockSpec((2, _R_NODE, hp2), lambda i: (0, i, 0)),
            pl.BlockSpec((2, _R_NODE, 128), lambda i: (0, i, 0)),
            pl.BlockSpec((d, 1), lambda i: (0, 0)),
            pl.BlockSpec((1, 1), lambda i: (0, 0)),
        ],
        out_specs=[pl.BlockSpec((_R_NODE, 1), lambda i: (i, 0))],
        out_shape=[jax.ShapeDtypeStruct((n, 1), f32)],
        compiler_params=_ARB,
    )(s_arr, cnt_arr, Wl, bl.reshape(1, 1))[0]


def _sc_gather(U, V, dst, src):
    n, h = U.shape
    hf = h // 16
    ept = N_EDGES // NW
    nch = ept // CHUNK

    mesh = plsc.VectorSubcoreMesh(**_SC_MESH)

    @functools.partial(
        pl.kernel,
        mesh=mesh,
        out_type=jax.ShapeDtypeStruct((N_EDGES, h), f32),
        scratch_types=[
            pltpu.VMEM((CHUNK,), jnp.int32),
            pltpu.VMEM((CHUNK,), jnp.int32),
            pltpu.VMEM((CHUNK,), jnp.int32),
            pltpu.VMEM((CHUNK,), jnp.int32),
            pltpu.VMEM((CHUNK, h), f32),
            pltpu.VMEM((CHUNK, h), f32),
            pltpu.VMEM((CHUNK, h), f32),
            pltpu.VMEM((CHUNK, h), f32),
            pltpu.SemaphoreType.DMA,
            pltpu.SemaphoreType.DMA,
            pltpu.SemaphoreType.DMA,
            pltpu.SemaphoreType.DMA,
            pltpu.SemaphoreType.DMA,
            pltpu.SemaphoreType.DMA,
        ],
    )
    def k(u_hbm, v_hbm, dst_hbm, src_hbm, h1_hbm,
          id0, is0, id1, is1, ra0, rb0, ra1, rb1,
          semi0, semi1, semr0, semr1, semw0, semw1):
        cid = lax.axis_index("c")
        sid = lax.axis_index("s")
        wid = sid * NC + cid
        base = wid * ept

        def off(c):
            return pl.multiple_of(base + c * CHUNK, 8)

        def issue_idx(c, id_, is_, sem):
            pltpu.async_copy(dst_hbm.at[pl.ds(off(c), CHUNK)], id_, sem)
            pltpu.async_copy(src_hbm.at[pl.ds(off(c), CHUNK)], is_, sem)

        def wait_idx(id_, is_, sem):
            pltpu.make_async_copy(dst_hbm.at[pl.ds(0, CHUNK)], id_, sem).wait()
            pltpu.make_async_copy(src_hbm.at[pl.ds(0, CHUNK)], is_, sem).wait()

        def issue_gather(id_, is_, ra_, rb_, sem):
            pltpu.async_copy(u_hbm.at[id_], ra_, sem)
            pltpu.async_copy(v_hbm.at[is_], rb_, sem)

        def wait_gather(id_, is_, ra_, rb_, sem):
            pltpu.make_async_copy(u_hbm.at[id_], ra_, sem).wait()
            pltpu.make_async_copy(v_hbm.at[is_], rb_, sem).wait()

        def compute(ra_, rb_):
            def rowfn(r, carry2):
                for f in range(hf):
                    sl = pl.ds(f * 16, 16)
                    plsc.addupdate(ra_.at[r, sl], rb_[r, sl])
                return carry2

            lax.fori_loop(0, CHUNK, rowfn, 0)

        def issue_write(c, ra_, sem):
            pltpu.async_copy(ra_, h1_hbm.at[pl.ds(off(c), CHUNK)], sem)

        def wait_write(ra_, sem):
            pltpu.make_async_copy(ra_, h1_hbm.at[pl.ds(0, CHUNK)], sem).wait()

        issue_idx(0, id0, is0, semi0)
        wait_idx(id0, is0, semi0)
        issue_gather(id0, is0, ra0, rb0, semr0)
        issue_idx(1, id1, is1, semi1)

        def pairbody(kp, carry):
            c1 = 2 * kp + 1
            c2 = 2 * kp + 2
            c3 = 2 * kp + 3
            wait_gather(id0, is0, ra0, rb0, semr0)
            compute(ra0, rb0)
            issue_write(2 * kp, ra0, semw0)

            @pl.when(c1 < nch)
            def _():
                wait_idx(id1, is1, semi1)

                @pl.when(kp > 0)
                def _():
                    wait_write(ra1, semw1)

                issue_gather(id1, is1, ra1, rb1, semr1)

                @pl.when(c2 < nch)
                def _():
                    issue_idx(c2, id0, is0, semi0)

            @pl.when(c1 < nch)
            def _():
                wait_gather(id1, is1, ra1, rb1, semr1)
                compute(ra1, rb1)
                issue_write(c1, ra1, semw1)

                @pl.when(c2 < nch)
                def _():
                    wait_idx(id0, is0, semi0)
                    wait_write(ra0, semw0)
                    issue_gather(id0, is0, ra0, rb0, semr0)

                    @pl.when(c3 < nch)
                    def _():
                        issue_idx(c3, id1, is1, semi1)

            return carry

        lax.fori_loop(0, (nch + 1) // 2, pairbody, 0)
        wait_write(ra0, semw0)
        wait_write(ra1, semw1)

    return k(U, V, dst, src)


def _sc_scatter(msg, dst, edge_split):
    if edge_split:
        ept = N_EDGES // 2 // NS
    else:
        ept = N_EDGES // NS
    nch = ept // CHUNK
    w = 128
    zrows = 80
    nblk = N_NODES // zrows

    mesh = plsc.VectorSubcoreMesh(**_SC_MESH)

    @functools.partial(
        pl.kernel,
        mesh=mesh,
        out_type=jax.ShapeDtypeStruct((NC, N_NODES, w), f32),
        scratch_types=[
            pltpu.VMEM((CHUNK,), jnp.int32),
            pltpu.VMEM((CHUNK,), jnp.int32),
            pltpu.VMEM((CHUNK, w), f32),
            pltpu.VMEM((CHUNK, w), f32),
            pltpu.VMEM((zrows, w), f32),
            pltpu.VMEM_SHARED((N_NODES, w), f32),
            pltpu.SemaphoreType.DMA,
            pltpu.SemaphoreType.DMA,
            pltpu.SemaphoreType.DMA,
            pltpu.SemaphoreType.DMA,
        ],
    )
    def k(msg_hbm, dst_hbm, s_hbm, ib0, ib1, bf0, bf1, zbuf, acc_sh,
          semL0, semL1, semS0, semS1):
        cid = lax.axis_index("c")
        sid = lax.axis_index("s")
        zero = jnp.zeros((16,), f32)

        def zf(r, carry):
            for f in range(w // 16):
                zbuf[r, pl.ds(f * 16, 16)] = zero
            return carry

        lax.fori_loop(0, zrows, zf, 0)
        nb = jnp.where(sid < nblk - (nblk // NS) * NS, nblk // NS + 1, nblk // NS)

        def zcopy(j, carry):
            off = pl.multiple_of((sid + j * NS) * zrows, 8)
            pltpu.sync_copy(zbuf, acc_sh.at[pl.ds(off, zrows)])
            return carry

        lax.fori_loop(0, nb, zcopy, 0)
        plsc.subcore_barrier()

        if edge_split:
            base = cid * (N_EDGES // 2) + sid * ept
        else:
            base = sid * ept

        def off(c):
            return pl.multiple_of(base + c * CHUNK, 8)

        def msg_slice(c):
            if edge_split:
                return msg_hbm.at[pl.ds(off(c), CHUNK)]
            return msg_hbm.at[cid, pl.ds(off(c), CHUNK)]

        def issue_load(c, ib, bf, sem):
            pltpu.async_copy(dst_hbm.at[pl.ds(off(c), CHUNK)], ib, sem)
            pltpu.async_copy(msg_slice(c), bf, sem)

        def wait_load(ib, bf, sem):
            pltpu.make_async_copy(dst_hbm.at[pl.ds(0, CHUNK)], ib, sem).wait()
            pltpu.make_async_copy(msg_slice(0), bf, sem).wait()

        def issue_scat(ib, bf, sem):
            pltpu.async_copy(bf, acc_sh.at[ib], sem, add=True)

        def wait_scat(ib, bf, sem):
            pltpu.make_async_copy(bf, acc_sh.at[ib], sem).wait()

        issue_load(0, ib0, bf0, semL0)
        issue_load(1, ib1, bf1, semL1)

        def pairbody(kp, carry):
            c1 = 2 * kp + 1
            c2 = 2 * kp + 2
            c3 = 2 * kp + 3
            wait_load(ib0, bf0, semL0)
            issue_scat(ib0, bf0, semS0)

            @pl.when(c1 < nch)
            def _():
                wait_load(ib1, bf1, semL1)
                issue_scat(ib1, bf1, semS1)

            @pl.when(c2 < nch)
            def _():
                wait_scat(ib0, bf0, semS0)
                issue_load(c2, ib0, bf0, semL0)

            @pl.when(c3 < nch)
            def _():
                wait_scat(ib1, bf1, semS1)
                issue_load(c3, ib1, bf1, semL1)

            return carry

        lax.fori_loop(0, (nch + 1) // 2, pairbody, 0)
        wait_scat(ib0, bf0, semS0)
        wait_scat(ib1, bf1, semS1)
        plsc.subcore_barrier()

        def outcopy(j, carry):
            off = pl.multiple_of((sid + j * NS) * zrows, 8)
            sl = pl.ds(off, zrows)
            pltpu.sync_copy(acc_sh.at[sl], s_hbm.at[cid, sl])
            return carry

        lax.fori_loop(0, nb, outcopy, 0)

    return k(msg, dst)


def _bn_affine(g, be, s, q):
    mu = s / N_EDGES
    var = q / N_EDGES - mu * mu
    a = g.reshape(1, -1) * lax.rsqrt(var + EPS)
    c = be.reshape(1, -1) - mu * a
    return a, c


def _pad_cols(m, w):
    return jnp.pad(m, ((0, 0), (0, w - m.shape[1])))


def _pad_rows(m, w):
    return jnp.pad(m, ((0, w - m.shape[0]), (0, 0)))


def _split_w1(p):
    w1 = p["W1"]
    d = w1.shape[1] // 2
    wa = w1[:, :d]
    wb = w1[:, d:]
    return (wa - wb).T, wb.T


def _layer_passes(h1, a1, c1, p, w2, wout, cnt_col, split):
    w3 = p["W3"].T
    s2, q2 = _mlp_stats(h1, a1, c1, w2, p["b2"])
    a2, c2 = _bn_affine(p["g2"], p["be2"], s2, q2)
    s3, q3 = _mlp2_stats(h1, a1, c1, w2, p["b2"], a2, c2, w3, p["b3"])
    a3, c3 = _bn_affine(p["g3"], p["be3"], s3, q3)
    return _mlp2_msg(h1, a1, c1, w2, p["b2"], a2, c2, w3, p["b3"],
                     a3, c3, wout, cnt_col, split)


def kernel(x, edge_index, edge_attr, params):
    del edge_attr
    src = edge_index[0].astype(jnp.int32)
    dst = edge_index[1].astype(jnp.int32)
    p0, p1, p2 = params["net0"], params["net1"], params["net2"]

    wd, wb = _split_w1(p0)
    u, v = _uv_from_x(x, _pad_cols(wd, 128), _pad_cols(wb, 128),
                      _pad_cols(p0["b1"].reshape(1, -1), 128).reshape(-1))
    h1 = _sc_gather(u, v, dst, src)
    s1, q1 = _stats(h1)
    a1, c1 = _bn_affine(p0["g1"], p0["be1"], s1[:, :64], q1[:, :64])
    msg = _layer_passes(h1, _pad_cols(a1, 128), _pad_cols(c1, 128), p0,
                        _pad_rows(p0["W2"].T, 128), 128,
                        cnt_col=True, split=False)
    out0 = _sc_scatter(msg, dst, edge_split=True)

    wd, wb = _split_w1(p1)
    u, v = _uv_sum_halves(out0, out0, wd, wb, p1["b1"], 64)
    h1 = _sc_gather(u, v, dst, src)
    s1, q1 = _stats(h1)
    a1, c1 = _bn_affine(p1["g1"], p1["be1"], s1, q1)
    msg = _layer_passes(h1, a1, c1, p1, p1["W2"].T, 128,
                        cnt_col=False, split=False)
    out1 = _sc_scatter(msg, dst, edge_split=True)

    wd, wb = _split_w1(p2)
    u, v = _uv_sum_halves(out1, out0, wd, wb, p2["b1"], 128)
    h1 = _sc_gather(u, v, dst, src)
    s1, q1 = _stats(h1)
    a1, c1 = _bn_affine(p2["g1"], p2["be1"], s1, q1)
    msg2 = _layer_passes(h1, a1, c1, p2, p2["W2"].T, 0,
                         cnt_col=False, split=True)
    out2 = _sc_scatter(msg2, dst, edge_split=False)

    return _head(out2, out0, params["lin"]["W"].T, params["lin"]["b"])


# speedup vs baseline: 1.0479x; 1.0020x over previous
import functools

import jax
import jax.numpy as jnp
from jax import lax
from jax.experimental import pallas as pl
from jax.experimental.pallas import tpu as pltpu
from jax.experimental.pallas import tpu_sc as plsc

N_NODES = 10000
N_EDGES = 320000
EPS = 1e-5

NC = 2
NS = 16
NW = NC * NS
CHUNK = 80
CNT_COL = 64

f32 = jnp.float32

_SC_MESH = dict(core_axis_name="c", subcore_axis_name="s")


_R_EDGE = 2560
_R_NODE = 2000

_ARB = pltpu.CompilerParams(dimension_semantics=("arbitrary",))


def _uv_from_x(x, Wd, Wb, b1):
    n, d = x.shape
    h = Wd.shape[1]

    def kern(x_ref, wd_ref, wb_ref, b1_ref, u_ref, v_ref):
        y = x_ref[...]
        u_ref[...] = jnp.dot(y, wd_ref[...], preferred_element_type=f32) + b1_ref[...]
        v_ref[...] = jnp.dot(y, wb_ref[...], preferred_element_type=f32)

    return pl.pallas_call(
        kern,
        grid=(n // _R_NODE,),
        in_specs=[
            pl.BlockSpec((_R_NODE, d), lambda i: (i, 0)),
            pl.BlockSpec((d, h), lambda i: (0, 0)),
            pl.BlockSpec((d, h), lambda i: (0, 0)),
            pl.BlockSpec((1, h), lambda i: (0, 0)),
        ],
        out_specs=[
            pl.BlockSpec((_R_NODE, h), lambda i: (i, 0)),
            pl.BlockSpec((_R_NODE, h), lambda i: (i, 0)),
        ],
        out_shape=[
            jax.ShapeDtypeStruct((n, h), f32),
            jax.ShapeDtypeStruct((n, h), f32),
        ],
        compiler_params=_ARB,
    )(x, Wd, Wb, b1.reshape(1, h))


def _uv_sum_halves(s_arr, cnt_arr, Wd, Wb, b1, d_use):
    _, n, w = s_arr.shape
    h = Wd.shape[1]

    def kern(s_ref, cnt_ref, wd_ref, wb_ref, b1_ref, u_ref, v_ref):
        cnt = (cnt_ref[0, :, CNT_COL:CNT_COL + 1]
               + cnt_ref[1, :, CNT_COL:CNT_COL + 1])
        inv = 1.0 / jnp.maximum(cnt, 1.0)
        tot = s_ref[0] + s_ref[1]
        y = tot[:, :d_use] * inv
        u_ref[...] = jnp.dot(y, wd_ref[...], preferred_element_type=f32) + b1_ref[...]
        v_ref[...] = jnp.dot(y, wb_ref[...], preferred_element_type=f32)

    return pl.pallas_call(
        kern,
        grid=(n // _R_NODE,),
        in_specs=[
            pl.BlockSpec((2, _R_NODE, w), lambda i: (0, i, 0)),
            pl.BlockSpec((2, _R_NODE, 128), lambda i: (0, i, 0)),
            pl.BlockSpec((d_use, h), lambda i: (0, 0)),
            pl.BlockSpec((d_use, h), lambda i: (0, 0)),
            pl.BlockSpec((1, h), lambda i: (0, 0)),
        ],
        out_specs=[
            pl.BlockSpec((_R_NODE, h), lambda i: (i, 0)),
            pl.BlockSpec((_R_NODE, h), lambda i: (i, 0)),
        ],
        out_shape=[
            jax.ShapeDtypeStruct((n, h), f32),
            jax.ShapeDtypeStruct((n, h), f32),
        ],
        compiler_params=_ARB,
    )(s_arr, cnt_arr, Wd, Wb, b1.reshape(1, h))


def _stats(h1):
    e, h = h1.shape

    def kern(h_ref, s_ref, q_ref):
        @pl.when(pl.program_id(0) == 0)
        def _():
            s_ref[...] = jnp.zeros_like(s_ref)
            q_ref[...] = jnp.zeros_like(q_ref)

        v = h_ref[...]
        s_ref[...] += jnp.sum(v, axis=0, keepdims=True)
        q_ref[...] += jnp.sum(v * v, axis=0, keepdims=True)

    return pl.pallas_call(
        kern,
        grid=(e // _R_EDGE,),
        in_specs=[pl.BlockSpec((_R_EDGE, h), lambda i: (i, 0))],
        out_specs=[
            pl.BlockSpec((1, h), lambda i: (0, 0)),
            pl.BlockSpec((1, h), lambda i: (0, 0)),
        ],
        out_shape=[
            jax.ShapeDtypeStruct((1, h), f32),
            jax.ShapeDtypeStruct((1, h), f32),
        ],
        compiler_params=_ARB,
    )(h1)


def _bdot(m, w):
    return jnp.dot(m, w, preferred_element_type=f32)


def _bn_params(s, q, g, be):
    mu = s * (1.0 / N_EDGES)
    var = q * (1.0 / N_EDGES) - mu * mu
    a = g * lax.rsqrt(var + EPS)
    c = be - mu * a
    return a, c


def _mlp_stats(h_in, s1, q1, g1, be1, W, b):
    e, hi = h_in.shape
    ho = W.shape[1]

    def kern(h_ref, s1_ref, q1_ref, g1_ref, be1_ref, w_ref, b_ref,
             s_ref, q_ref):
        @pl.when(pl.program_id(0) == 0)
        def _():
            s_ref[...] = jnp.zeros_like(s_ref)
            q_ref[...] = jnp.zeros_like(q_ref)

        a1, c1 = _bn_params(s1_ref[...], q1_ref[...], g1_ref[...], be1_ref[...])
        m = jnp.maximum(h_ref[...] * a1 + c1, 0.0)
        v = _bdot(m, w_ref[...]) + b_ref[...]
        s_ref[...] += jnp.sum(v, axis=0, keepdims=True)
        q_ref[...] += jnp.sum(v * v, axis=0, keepdims=True)

    vec = lambda w: pl.BlockSpec((1, w), lambda i: (0, 0))
    return pl.pallas_call(
        kern,
        grid=(e // _R_EDGE,),
        in_specs=[
            pl.BlockSpec((_R_EDGE, hi), lambda i: (i, 0)),
            vec(hi), vec(hi), vec(hi), vec(hi),
            pl.BlockSpec((hi, ho), lambda i: (0, 0)),
            vec(ho),
        ],
        out_specs=[vec(ho), vec(ho)],
        out_shape=[
            jax.ShapeDtypeStruct((1, ho), f32),
            jax.ShapeDtypeStruct((1, ho), f32),
        ],
        compiler_params=_ARB,
    )(h_in, s1, q1, g1, be1, W, b.reshape(1, ho))


def _mlp2_stats(h_in, s1, q1, g1, be1, W2, b2, s2, q2, g2, be2, W3, b3):
    e, hi = h_in.shape
    hm = W2.shape[1]
    ho = W3.shape[1]

    def kern(h_ref, s1_ref, q1_ref, g1_ref, be1_ref, w2_ref, b2_ref,
             s2_ref, q2_ref, g2_ref, be2_ref, w3_ref, b3_ref, s_ref, q_ref):
        @pl.when(pl.program_id(0) == 0)
        def _():
            s_ref[...] = jnp.zeros_like(s_ref)
            q_ref[...] = jnp.zeros_like(q_ref)

        a1, c1 = _bn_params(s1_ref[...], q1_ref[...], g1_ref[...], be1_ref[...])
        a2, c2 = _bn_params(s2_ref[...], q2_ref[...], g2_ref[...], be2_ref[...])
        m1 = jnp.maximum(h_ref[...] * a1 + c1, 0.0)
        h2 = _bdot(m1, w2_ref[...]) + b2_ref[...]
        m2 = jnp.maximum(h2 * a2 + c2, 0.0)
        v = _bdot(m2, w3_ref[...]) + b3_ref[...]
        s_ref[...] += jnp.sum(v, axis=0, keepdims=True)
        q_ref[...] += jnp.sum(v * v, axis=0, keepdims=True)

    vec = lambda w: pl.BlockSpec((1, w), lambda i: (0, 0))
    return pl.pallas_call(
        kern,
        grid=(e // _R_EDGE,),
        in_specs=[
            pl.BlockSpec((_R_EDGE, hi), lambda i: (i, 0)),
            vec(hi), vec(hi), vec(hi), vec(hi),
            pl.BlockSpec((hi, hm), lambda i: (0, 0)),
            vec(hm),
            vec(hm), vec(hm), vec(hm), vec(hm),
            pl.BlockSpec((hm, ho), lambda i: (0, 0)),
            vec(ho),
        ],
        out_specs=[vec(ho), vec(ho)],
        out_shape=[
            jax.ShapeDtypeStruct((1, ho), f32),
            jax.ShapeDtypeStruct((1, ho), f32),
        ],
        compiler_params=_ARB,
    )(h_in, s1, q1, g1, be1, W2, b2.reshape(1, hm),
      s2, q2, g2, be2, W3, b3.reshape(1, ho))


def _mlp2_msg(h_in, s1, q1, g1, be1, W2, b2, s2, q2, g2, be2, W3, b3,
              s3, q3, g3, be3, wout, cnt_col, split):
    e, hi = h_in.shape
    hm = W2.shape[1]
    ho = W3.shape[1]
    extra = 0 if split else wout - ho
    h2o = ho // 2

    def kern(h_ref, s1_ref, q1_ref, g1_ref, be1_ref, w2_ref, b2_ref,
             s2_ref, q2_ref, g2_ref, be2_ref, w3_ref, b3_ref,
             s3_ref, q3_ref, g3_ref, be3_ref, o_ref):
        a1, c1 = _bn_params(s1_ref[...], q1_ref[...], g1_ref[...], be1_ref[...])
        a2, c2 = _bn_params(s2_ref[...], q2_ref[...], g2_ref[...], be2_ref[...])
        a3, c3 = _bn_params(s3_ref[...], q3_ref[...], g3_ref[...], be3_ref[...])
        m1 = jnp.maximum(h_ref[...] * a1 + c1, 0.0)
        h2 = _bdot(m1, w2_ref[...]) + b2_ref[...]
        m2 = jnp.maximum(h2 * a2 + c2, 0.0)
        h3 = _bdot(m2, w3_ref[...]) + b3_ref[...]
        m = jnp.maximum(h3 * a3 + c3, 0.0)
        if split:
            o_ref[0] = m[:, :h2o]
            o_ref[1] = m[:, h2o:]
        elif extra == 0:
            o_ref[...] = m
        else:
            if cnt_col:
                col = lax.broadcasted_iota(jnp.int32, (_R_EDGE, extra), 1)
                pad = jnp.where(col == 0, 1.0, 0.0).astype(f32)
            else:
                pad = jnp.zeros((_R_EDGE, extra), f32)
            o_ref[...] = jnp.concatenate([m, pad], axis=-1)

    if split:
        out_spec = [pl.BlockSpec((2, _R_EDGE, h2o), lambda i: (0, i, 0))]
        out_shape = [jax.ShapeDtypeStruct((2, e, h2o), f32)]
    else:
        out_spec = [pl.BlockSpec((_R_EDGE, wout), lambda i: (i, 0))]
        out_shape = [jax.ShapeDtypeStruct((e, wout), f32)]

    vec = lambda w: pl.BlockSpec((1, w), lambda i: (0, 0))
    return pl.pallas_call(
        kern,
        grid=(e // _R_EDGE,),
        in_specs=[
            pl.BlockSpec((_R_EDGE, hi), lambda i: (i, 0)),
            vec(hi), vec(hi), vec(hi), vec(hi),
            pl.BlockSpec((hi, hm), lambda i: (0, 0)),
            vec(hm),
            vec(hm), vec(hm), vec(hm), vec(hm),
            pl.BlockSpec((hm, ho), lambda i: (0, 0)),
            vec(ho),
            vec(ho), vec(ho), vec(ho), vec(ho),
        ],
        out_specs=out_spec,
        out_shape=out_shape,
        compiler_params=_ARB,
    )(h_in, s1, q1, g1, be1, W2, b2.reshape(1, hm),
      s2, q2, g2, be2, W3, b3.reshape(1, ho), s3, q3, g3, be3)[0]


def _head(s_arr, cnt_arr, Wl, bl):
    _, n, hp2 = s_arr.shape
    d = 2 * hp2

    def kern(s_ref, cnt_ref, w_ref, b_ref, o_ref):
        cnt = (cnt_ref[0, :, CNT_COL:CNT_COL + 1]
               + cnt_ref[1, :, CNT_COL:CNT_COL + 1])
        inv = 1.0 / jnp.maximum(cnt, 1.0)
        y = jnp.concatenate([s_ref[0], s_ref[1]], axis=-1) * inv
        z = jnp.dot(y, w_ref[...], preferred_element_type=f32) + b_ref[...]
        o_ref[...] = jax.nn.sigmoid(z)

    return pl.pallas_call(
        kern,
        grid=(n // _R_NODE,),
        in_specs=[
            pl.BlockSpec((2, _R_NODE, hp2), lambda i: (0, i, 0)),
            pl.BlockSpec((2, _R_NODE, 128), lambda i: (0, i, 0)),
            pl.BlockSpec((d, 1), lambda i: (0, 0)),
            pl.BlockSpec((1, 1), lambda i: (0, 0)),
        ],
        out_specs=[pl.BlockSpec((_R_NODE, 1), lambda i: (i, 0))],
        out_shape=[jax.ShapeDtypeStruct((n, 1), f32)],
        compiler_params=_ARB,
    )(s_arr, cnt_arr, Wl, bl.reshape(1, 1))[0]


def _sc_gather(U, V, dst, src):
    n, h = U.shape
    hf = h // 16
    ept = N_EDGES // NW
    nch = ept // CHUNK

    mesh = plsc.VectorSubcoreMesh(**_SC_MESH)

    @functools.partial(
        pl.kernel,
        mesh=mesh,
        out_type=jax.ShapeDtypeStruct((N_EDGES, h), f32),
        scratch_types=[
            pltpu.VMEM((CHUNK,), jnp.int32),
            pltpu.VMEM((CHUNK,), jnp.int32),
            pltpu.VMEM((CHUNK,), jnp.int32),
            pltpu.VMEM((CHUNK,), jnp.int32),
            pltpu.VMEM((CHUNK, h), f32),
            pltpu.VMEM((CHUNK, h), f32),
            pltpu.VMEM((CHUNK, h), f32),
            pltpu.VMEM((CHUNK, h), f32),
            pltpu.SemaphoreType.DMA,
            pltpu.SemaphoreType.DMA,
            pltpu.SemaphoreType.DMA,
            pltpu.SemaphoreType.DMA,
            pltpu.SemaphoreType.DMA,
            pltpu.SemaphoreType.DMA,
        ],
    )
    def k(u_hbm, v_hbm, dst_hbm, src_hbm, h1_hbm,
          id0, is0, id1, is1, ra0, rb0, ra1, rb1,
          semi0, semi1, semr0, semr1, semw0, semw1):
        cid = lax.axis_index("c")
        sid = lax.axis_index("s")
        wid = sid * NC + cid
        base = wid * ept

        def off(c):
            return pl.multiple_of(base + c * CHUNK, 8)

        def issue_idx(c, id_, is_, sem):
            pltpu.async_copy(dst_hbm.at[pl.ds(off(c), CHUNK)], id_, sem)
            pltpu.async_copy(src_hbm.at[pl.ds(off(c), CHUNK)], is_, sem)

        def wait_idx(id_, is_, sem):
            pltpu.make_async_copy(dst_hbm.at[pl.ds(0, CHUNK)], id_, sem).wait()
            pltpu.make_async_copy(src_hbm.at[pl.ds(0, CHUNK)], is_, sem).wait()

        def issue_gather(id_, is_, ra_, rb_, sem):
            pltpu.async_copy(u_hbm.at[id_], ra_, sem)
            pltpu.async_copy(v_hbm.at[is_], rb_, sem)

        def wait_gather(id_, is_, ra_, rb_, sem):
            pltpu.make_async_copy(u_hbm.at[id_], ra_, sem).wait()
            pltpu.make_async_copy(v_hbm.at[is_], rb_, sem).wait()

        def compute(ra_, rb_):
            def rowfn(r, carry2):
                for f in range(hf):
                    sl = pl.ds(f * 16, 16)
                    plsc.addupdate(ra_.at[r, sl], rb_[r, sl])
                return carry2

            lax.fori_loop(0, CHUNK, rowfn, 0)

        def issue_write(c, ra_, sem):
            pltpu.async_copy(ra_, h1_hbm.at[pl.ds(off(c), CHUNK)], sem)

        def wait_write(ra_, sem):
            pltpu.make_async_copy(ra_, h1_hbm.at[pl.ds(0, CHUNK)], sem).wait()

        issue_idx(0, id0, is0, semi0)
        wait_idx(id0, is0, semi0)
        issue_gather(id0, is0, ra0, rb0, semr0)
        issue_idx(1, id1, is1, semi1)

        def pairbody(kp, carry):
            c1 = 2 * kp + 1
            c2 = 2 * kp + 2
            c3 = 2 * kp + 3
            wait_gather(id0, is0, ra0, rb0, semr0)
            compute(ra0, rb0)
            issue_write(2 * kp, ra0, semw0)

            @pl.when(c1 < nch)
            def _():
                wait_idx(id1, is1, semi1)

                @pl.when(kp > 0)
                def _():
                    wait_write(ra1, semw1)

                issue_gather(id1, is1, ra1, rb1, semr1)

                @pl.when(c2 < nch)
                def _():
                    issue_idx(c2, id0, is0, semi0)

            @pl.when(c1 < nch)
            def _():
                wait_gather(id1, is1, ra1, rb1, semr1)
                compute(ra1, rb1)
                issue_write(c1, ra1, semw1)

                @pl.when(c2 < nch)
                def _():
                    wait_idx(id0, is0, semi0)
                    wait_write(ra0, semw0)
                    issue_gather(id0, is0, ra0, rb0, semr0)

                    @pl.when(c3 < nch)
                    def _():
                        issue_idx(c3, id1, is1, semi1)

            return carry

        lax.fori_loop(0, (nch + 1) // 2, pairbody, 0)
        wait_write(ra0, semw0)
        wait_write(ra1, semw1)

    return k(U, V, dst, src)


def _sc_scatter(msg, dst, edge_split):
    if edge_split:
        ept = N_EDGES // 2 // NS
    else:
        ept = N_EDGES // NS
    nch = ept // CHUNK
    w = 128
    zrows = 80
    nblk = N_NODES // zrows

    mesh = plsc.VectorSubcoreMesh(**_SC_MESH)

    @functools.partial(
        pl.kernel,
        mesh=mesh,
        out_type=jax.ShapeDtypeStruct((NC, N_NODES, w), f32),
        scratch_types=[
            pltpu.VMEM((CHUNK,), jnp.int32),
            pltpu.VMEM((CHUNK,), jnp.int32),
            pltpu.VMEM((CHUNK, w), f32),
            pltpu.VMEM((CHUNK, w), f32),
            pltpu.VMEM((zrows, w), f32),
            pltpu.VMEM_SHARED((N_NODES, w), f32),
            pltpu.SemaphoreType.DMA,
            pltpu.SemaphoreType.DMA,
            pltpu.SemaphoreType.DMA,
            pltpu.SemaphoreType.DMA,
        ],
    )
    def k(msg_hbm, dst_hbm, s_hbm, ib0, ib1, bf0, bf1, zbuf, acc_sh,
          semL0, semL1, semS0, semS1):
        cid = lax.axis_index("c")
        sid = lax.axis_index("s")
        zero = jnp.zeros((16,), f32)

        def zf(r, carry):
            for f in range(w // 16):
                zbuf[r, pl.ds(f * 16, 16)] = zero
            return carry

        lax.fori_loop(0, zrows, zf, 0)
        nb = jnp.where(sid < nblk - (nblk // NS) * NS, nblk // NS + 1, nblk // NS)

        def zcopy(j, carry):
            off = pl.multiple_of((sid + j * NS) * zrows, 8)
            pltpu.sync_copy(zbuf, acc_sh.at[pl.ds(off, zrows)])
            return carry

        lax.fori_loop(0, nb, zcopy, 0)
        plsc.subcore_barrier()

        if edge_split:
            base = cid * (N_EDGES // 2) + sid * ept
        else:
            base = sid * ept

        def off(c):
            return pl.multiple_of(base + c * CHUNK, 8)

        def msg_slice(c):
            if edge_split:
                return msg_hbm.at[pl.ds(off(c), CHUNK)]
            return msg_hbm.at[cid, pl.ds(off(c), CHUNK)]

        def issue_load(c, ib, bf, sem):
            pltpu.async_copy(dst_hbm.at[pl.ds(off(c), CHUNK)], ib, sem)
            pltpu.async_copy(msg_slice(c), bf, sem)

        def wait_load(ib, bf, sem):
            pltpu.make_async_copy(dst_hbm.at[pl.ds(0, CHUNK)], ib, sem).wait()
            pltpu.make_async_copy(msg_slice(0), bf, sem).wait()

        def issue_scat(ib, bf, sem):
            pltpu.async_copy(bf, acc_sh.at[ib], sem, add=True)

        def wait_scat(ib, bf, sem):
            pltpu.make_async_copy(bf, acc_sh.at[ib], sem).wait()

        issue_load(0, ib0, bf0, semL0)
        issue_load(1, ib1, bf1, semL1)

        def pairbody(kp, carry):
            c1 = 2 * kp + 1
            c2 = 2 * kp + 2
            c3 = 2 * kp + 3
            wait_load(ib0, bf0, semL0)
            issue_scat(ib0, bf0, semS0)

            @pl.when(c1 < nch)
            def _():
                wait_load(ib1, bf1, semL1)
                issue_scat(ib1, bf1, semS1)

            @pl.when(c2 < nch)
            def _():
                wait_scat(ib0, bf0, semS0)
                issue_load(c2, ib0, bf0, semL0)

            @pl.when(c3 < nch)
            def _():
                wait_scat(ib1, bf1, semS1)
                issue_load(c3, ib1, bf1, semL1)

            return carry

        lax.fori_loop(0, (nch + 1) // 2, pairbody, 0)
        wait_scat(ib0, bf0, semS0)
        wait_scat(ib1, bf1, semS1)
        plsc.subcore_barrier()

        def outcopy(j, carry):
            off = pl.multiple_of((sid + j * NS) * zrows, 8)
            sl = pl.ds(off, zrows)
            pltpu.sync_copy(acc_sh.at[sl], s_hbm.at[cid, sl])
            return carry

        lax.fori_loop(0, nb, outcopy, 0)

    return k(msg, dst)


def _pad_cols(m, w):
    return jnp.pad(m, ((0, 0), (0, w - m.shape[1])))


def _pad_rows(m, w):
    return jnp.pad(m, ((0, w - m.shape[0]), (0, 0)))


def _split_w1(p):
    w1 = p["W1"]
    d = w1.shape[1] // 2
    wa = w1[:, :d]
    wb = w1[:, d:]
    return (wa - wb).T, wb.T


def _layer_passes(h1, s1, q1, g1, be1, p, w2, wout, cnt_col, split):
    w3 = p["W3"].T
    r = lambda v: v.reshape(1, -1)
    g2, be2 = r(p["g2"]), r(p["be2"])
    g3, be3 = r(p["g3"]), r(p["be3"])
    s2, q2 = _mlp_stats(h1, s1, q1, g1, be1, w2, p["b2"])
    s3, q3 = _mlp2_stats(h1, s1, q1, g1, be1, w2, p["b2"],
                         s2, q2, g2, be2, w3, p["b3"])
    return _mlp2_msg(h1, s1, q1, g1, be1, w2, p["b2"],
                     s2, q2, g2, be2, w3, p["b3"],
                     s3, q3, g3, be3, wout, cnt_col, split)


def kernel(x, edge_index, edge_attr, params):
    del edge_attr
    src = edge_index[0].astype(jnp.int32)
    dst = edge_index[1].astype(jnp.int32)
    p0, p1, p2 = params["net0"], params["net1"], params["net2"]

    wd, wb = _split_w1(p0)
    u, v = _uv_from_x(x, _pad_cols(wd, 128), _pad_cols(wb, 128),
                      _pad_cols(p0["b1"].reshape(1, -1), 128).reshape(-1))
    h1 = _sc_gather(u, v, dst, src)
    s1, q1 = _stats(h1)
    g1p = _pad_cols(p0["g1"].reshape(1, -1), 128)
    be1p = _pad_cols(p0["be1"].reshape(1, -1), 128)
    msg = _layer_passes(h1, s1, q1, g1p, be1p, p0,
                        _pad_rows(p0["W2"].T, 128), 128,
                        cnt_col=True, split=False)
    out0 = _sc_scatter(msg, dst, edge_split=True)

    wd, wb = _split_w1(p1)
    u, v = _uv_sum_halves(out0, out0, wd, wb, p1["b1"], 64)
    h1 = _sc_gather(u, v, dst, src)
    s1, q1 = _stats(h1)
    msg = _layer_passes(h1, s1, q1, p1["g1"].reshape(1, -1),
                        p1["be1"].reshape(1, -1), p1, p1["W2"].T, 128,
                        cnt_col=False, split=False)
    out1 = _sc_scatter(msg, dst, edge_split=True)

    wd, wb = _split_w1(p2)
    u, v = _uv_sum_halves(out1, out0, wd, wb, p2["b1"], 128)
    h1 = _sc_gather(u, v, dst, src)
    s1, q1 = _stats(h1)
    msg2 = _layer_passes(h1, s1, q1, p2["g1"].reshape(1, -1),
                         p2["be1"].reshape(1, -1), p2, p2["W2"].T, 0,
                         cnt_col=False, split=True)
    out2 = _sc_scatter(msg2, dst, edge_split=False)

    return _head(out2, out0, params["lin"]["W"].T, params["lin"]["b"])


# speedup vs baseline: 1.0811x; 1.0317x over previous
import functools

import jax
import jax.numpy as jnp
from jax import lax
from jax.experimental import pallas as pl
from jax.experimental.pallas import tpu as pltpu
from jax.experimental.pallas import tpu_sc as plsc

N_NODES = 10000
N_EDGES = 320000
EPS = 1e-5

NC = 2
NS = 16
NW = NC * NS
CHUNK = 80
CNT_COL = 64

f32 = jnp.float32

_SC_MESH = dict(core_axis_name="c", subcore_axis_name="s")


_R_EDGE = 2560
_R_NODE = 2000

_ARB = pltpu.CompilerParams(dimension_semantics=("arbitrary",))


def _uv_from_x(x, Wd, Wb, b1):
    n, d = x.shape
    h = Wd.shape[1]

    def kern(x_ref, wd_ref, wb_ref, b1_ref, u_ref, v_ref):
        y = x_ref[...]
        u_ref[...] = jnp.dot(y, wd_ref[...], preferred_element_type=f32) + b1_ref[...]
        v_ref[...] = jnp.dot(y, wb_ref[...], preferred_element_type=f32)

    return pl.pallas_call(
        kern,
        grid=(n // _R_NODE,),
        in_specs=[
            pl.BlockSpec((_R_NODE, d), lambda i: (i, 0)),
            pl.BlockSpec((d, h), lambda i: (0, 0)),
            pl.BlockSpec((d, h), lambda i: (0, 0)),
            pl.BlockSpec((1, h), lambda i: (0, 0)),
        ],
        out_specs=[
            pl.BlockSpec((_R_NODE, h), lambda i: (i, 0)),
            pl.BlockSpec((_R_NODE, h), lambda i: (i, 0)),
        ],
        out_shape=[
            jax.ShapeDtypeStruct((n, h), f32),
            jax.ShapeDtypeStruct((n, h), f32),
        ],
        compiler_params=_ARB,
    )(x, Wd, Wb, b1.reshape(1, h))


def _uv_sum_halves(s_arr, cnt_arr, Wd, Wb, b1, d_use):
    _, n, w = s_arr.shape
    h = Wd.shape[1]

    def kern(s_ref, cnt_ref, wd_ref, wb_ref, b1_ref, u_ref, v_ref):
        cnt = (cnt_ref[0, :, CNT_COL:CNT_COL + 1]
               + cnt_ref[1, :, CNT_COL:CNT_COL + 1])
        inv = 1.0 / jnp.maximum(cnt, 1.0)
        tot = s_ref[0] + s_ref[1]
        y = tot[:, :d_use] * inv
        u_ref[...] = jnp.dot(y, wd_ref[...], preferred_element_type=f32) + b1_ref[...]
        v_ref[...] = jnp.dot(y, wb_ref[...], preferred_element_type=f32)

    return pl.pallas_call(
        kern,
        grid=(n // _R_NODE,),
        in_specs=[
            pl.BlockSpec((2, _R_NODE, w), lambda i: (0, i, 0)),
            pl.BlockSpec((2, _R_NODE, 128), lambda i: (0, i, 0)),
            pl.BlockSpec((d_use, h), lambda i: (0, 0)),
            pl.BlockSpec((d_use, h), lambda i: (0, 0)),
            pl.BlockSpec((1, h), lambda i: (0, 0)),
        ],
        out_specs=[
            pl.BlockSpec((_R_NODE, h), lambda i: (i, 0)),
            pl.BlockSpec((_R_NODE, h), lambda i: (i, 0)),
        ],
        out_shape=[
            jax.ShapeDtypeStruct((n, h), f32),
            jax.ShapeDtypeStruct((n, h), f32),
        ],
        compiler_params=_ARB,
    )(s_arr, cnt_arr, Wd, Wb, b1.reshape(1, h))


def _stats(h1):
    e, h = h1.shape

    def kern(h_ref, s_ref, q_ref):
        @pl.when(pl.program_id(0) == 0)
        def _():
            s_ref[...] = jnp.zeros_like(s_ref)
            q_ref[...] = jnp.zeros_like(q_ref)

        v = h_ref[...]
        s_ref[...] += jnp.sum(v, axis=0, keepdims=True)
        q_ref[...] += jnp.sum(v * v, axis=0, keepdims=True)

    return pl.pallas_call(
        kern,
        grid=(e // _R_EDGE,),
        in_specs=[pl.BlockSpec((_R_EDGE, h), lambda i: (i, 0))],
        out_specs=[
            pl.BlockSpec((1, h), lambda i: (0, 0)),
            pl.BlockSpec((1, h), lambda i: (0, 0)),
        ],
        out_shape=[
            jax.ShapeDtypeStruct((1, h), f32),
            jax.ShapeDtypeStruct((1, h), f32),
        ],
        compiler_params=_ARB,
    )(h1)


def _bdot(m, w):
    return jnp.dot(m, w, preferred_element_type=f32)


def _bn_params(s, q, g, be):
    mu = s * (1.0 / N_EDGES)
    var = q * (1.0 / N_EDGES) - mu * mu
    a = g * lax.rsqrt(var + EPS)
    c = be - mu * a
    return a, c


def _mlp_stats(h_in, s1, q1, g1, be1, W, b):
    e, hi = h_in.shape
    ho = W.shape[1]

    def kern(h_ref, s1_ref, q1_ref, g1_ref, be1_ref, w_ref, b_ref,
             s_ref, q_ref):
        @pl.when(pl.program_id(0) == 0)
        def _():
            s_ref[...] = jnp.zeros_like(s_ref)
            q_ref[...] = jnp.zeros_like(q_ref)

        a1, c1 = _bn_params(s1_ref[...], q1_ref[...], g1_ref[...], be1_ref[...])
        m = jnp.maximum(h_ref[...] * a1 + c1, 0.0)
        v = _bdot(m, w_ref[...]) + b_ref[...]
        s_ref[...] += jnp.sum(v, axis=0, keepdims=True)
        q_ref[...] += jnp.sum(v * v, axis=0, keepdims=True)

    vec = lambda w: pl.BlockSpec((1, w), lambda i: (0, 0))
    return pl.pallas_call(
        kern,
        grid=(e // _R_EDGE,),
        in_specs=[
            pl.BlockSpec((_R_EDGE, hi), lambda i: (i, 0)),
            vec(hi), vec(hi), vec(hi), vec(hi),
            pl.BlockSpec((hi, ho), lambda i: (0, 0)),
            vec(ho),
        ],
        out_specs=[vec(ho), vec(ho)],
        out_shape=[
            jax.ShapeDtypeStruct((1, ho), f32),
            jax.ShapeDtypeStruct((1, ho), f32),
        ],
        compiler_params=_ARB,
    )(h_in, s1, q1, g1, be1, W, b.reshape(1, ho))


def _mlp2_stats(h_in, s1, q1, g1, be1, W2, b2, s2, q2, g2, be2, W3, b3):
    e, hi = h_in.shape
    hm = W2.shape[1]
    ho = W3.shape[1]

    def kern(h_ref, s1_ref, q1_ref, g1_ref, be1_ref, w2_ref, b2_ref,
             s2_ref, q2_ref, g2_ref, be2_ref, w3_ref, b3_ref, s_ref, q_ref):
        @pl.when(pl.program_id(0) == 0)
        def _():
            s_ref[...] = jnp.zeros_like(s_ref)
            q_ref[...] = jnp.zeros_like(q_ref)

        a1, c1 = _bn_params(s1_ref[...], q1_ref[...], g1_ref[...], be1_ref[...])
        a2, c2 = _bn_params(s2_ref[...], q2_ref[...], g2_ref[...], be2_ref[...])
        m1 = jnp.maximum(h_ref[...] * a1 + c1, 0.0)
        h2 = _bdot(m1, w2_ref[...]) + b2_ref[...]
        m2 = jnp.maximum(h2 * a2 + c2, 0.0)
        v = _bdot(m2, w3_ref[...]) + b3_ref[...]
        s_ref[...] += jnp.sum(v, axis=0, keepdims=True)
        q_ref[...] += jnp.sum(v * v, axis=0, keepdims=True)

    vec = lambda w: pl.BlockSpec((1, w), lambda i: (0, 0))
    return pl.pallas_call(
        kern,
        grid=(e // _R_EDGE,),
        in_specs=[
            pl.BlockSpec((_R_EDGE, hi), lambda i: (i, 0)),
            vec(hi), vec(hi), vec(hi), vec(hi),
            pl.BlockSpec((hi, hm), lambda i: (0, 0)),
            vec(hm),
            vec(hm), vec(hm), vec(hm), vec(hm),
            pl.BlockSpec((hm, ho), lambda i: (0, 0)),
            vec(ho),
        ],
        out_specs=[vec(ho), vec(ho)],
        out_shape=[
            jax.ShapeDtypeStruct((1, ho), f32),
            jax.ShapeDtypeStruct((1, ho), f32),
        ],
        compiler_params=_ARB,
    )(h_in, s1, q1, g1, be1, W2, b2.reshape(1, hm),
      s2, q2, g2, be2, W3, b3.reshape(1, ho))


def _mlp2_msg(h_in, s1, q1, g1, be1, W2, b2, s2, q2, g2, be2, W3, b3,
              s3, q3, g3, be3, wout, cnt_col, split):
    e, hi = h_in.shape
    hm = W2.shape[1]
    ho = W3.shape[1]
    extra = 0 if split else wout - ho
    h2o = ho // 2

    def kern(h_ref, s1_ref, q1_ref, g1_ref, be1_ref, w2_ref, b2_ref,
             s2_ref, q2_ref, g2_ref, be2_ref, w3_ref, b3_ref,
             s3_ref, q3_ref, g3_ref, be3_ref, o_ref):
        a1, c1 = _bn_params(s1_ref[...], q1_ref[...], g1_ref[...], be1_ref[...])
        a2, c2 = _bn_params(s2_ref[...], q2_ref[...], g2_ref[...], be2_ref[...])
        a3, c3 = _bn_params(s3_ref[...], q3_ref[...], g3_ref[...], be3_ref[...])
        m1 = jnp.maximum(h_ref[...] * a1 + c1, 0.0)
        h2 = _bdot(m1, w2_ref[...]) + b2_ref[...]
        m2 = jnp.maximum(h2 * a2 + c2, 0.0)
        h3 = _bdot(m2, w3_ref[...]) + b3_ref[...]
        m = jnp.maximum(h3 * a3 + c3, 0.0)
        if split:
            o_ref[0] = m[:, :h2o]
            o_ref[1] = m[:, h2o:]
        elif extra == 0:
            o_ref[...] = m
        else:
            if cnt_col:
                col = lax.broadcasted_iota(jnp.int32, (_R_EDGE, extra), 1)
                pad = jnp.where(col == 0, 1.0, 0.0).astype(f32)
            else:
                pad = jnp.zeros((_R_EDGE, extra), f32)
            o_ref[...] = jnp.concatenate([m, pad], axis=-1)

    if split:
        out_spec = [pl.BlockSpec((2, _R_EDGE, h2o), lambda i: (0, i, 0))]
        out_shape = [jax.ShapeDtypeStruct((2, e, h2o), f32)]
    else:
        out_spec = [pl.BlockSpec((_R_EDGE, wout), lambda i: (i, 0))]
        out_shape = [jax.ShapeDtypeStruct((e, wout), f32)]

    vec = lambda w: pl.BlockSpec((1, w), lambda i: (0, 0))
    return pl.pallas_call(
        kern,
        grid=(e // _R_EDGE,),
        in_specs=[
            pl.BlockSpec((_R_EDGE, hi), lambda i: (i, 0)),
            vec(hi), vec(hi), vec(hi), vec(hi),
            pl.BlockSpec((hi, hm), lambda i: (0, 0)),
            vec(hm),
            vec(hm), vec(hm), vec(hm), vec(hm),
            pl.BlockSpec((hm, ho), lambda i: (0, 0)),
            vec(ho),
            vec(ho), vec(ho), vec(ho), vec(ho),
        ],
        out_specs=out_spec,
        out_shape=out_shape,
        compiler_params=_ARB,
    )(h_in, s1, q1, g1, be1, W2, b2.reshape(1, hm),
      s2, q2, g2, be2, W3, b3.reshape(1, ho), s3, q3, g3, be3)[0]


def _head(s_arr, cnt_arr, Wl, bl):
    _, n, hp2 = s_arr.shape
    d = 2 * hp2

    def kern(s_ref, cnt_ref, w_ref, b_ref, o_ref):
        cnt = (cnt_ref[0, :, CNT_COL:CNT_COL + 1]
               + cnt_ref[1, :, CNT_COL:CNT_COL + 1])
        inv = 1.0 / jnp.maximum(cnt, 1.0)
        y = jnp.concatenate([s_ref[0], s_ref[1]], axis=-1) * inv
        z = jnp.dot(y, w_ref[...], preferred_element_type=f32) + b_ref[...]
        o_ref[...] = jax.nn.sigmoid(z)

    return pl.pallas_call(
        kern,
        grid=(n // _R_NODE,),
        in_specs=[
            pl.BlockSpec((2, _R_NODE, hp2), lambda i: (0, i, 0)),
            pl.BlockSpec((2, _R_NODE, 128), lambda i: (0, i, 0)),
            pl.BlockSpec((d, 1), lambda i: (0, 0)),
            pl.BlockSpec((1, 1), lambda i: (0, 0)),
        ],
        out_specs=[pl.BlockSpec((_R_NODE, 1), lambda i: (i, 0))],
        out_shape=[jax.ShapeDtypeStruct((n, 1), f32)],
        compiler_params=_ARB,
    )(s_arr, cnt_arr, Wl, bl.reshape(1, 1))[0]


def _sc_gather(U, V, dst, src):
    n, h = U.shape
    hf = h // 16
    ck = 128 if h <= 128 else 80
    total = N_EDGES // ck
    nbase, rem = divmod(total, NW)

    mesh = plsc.VectorSubcoreMesh(**_SC_MESH)

    @functools.partial(
        pl.kernel,
        mesh=mesh,
        out_type=jax.ShapeDtypeStruct((N_EDGES, h), f32),
        scratch_types=[
            pltpu.VMEM((ck,), jnp.int32),
            pltpu.VMEM((ck,), jnp.int32),
            pltpu.VMEM((ck,), jnp.int32),
            pltpu.VMEM((ck,), jnp.int32),
            pltpu.VMEM((ck, h), f32),
            pltpu.VMEM((ck, h), f32),
            pltpu.VMEM((ck, h), f32),
            pltpu.VMEM((ck, h), f32),
            pltpu.SemaphoreType.DMA,
            pltpu.SemaphoreType.DMA,
            pltpu.SemaphoreType.DMA,
            pltpu.SemaphoreType.DMA,
            pltpu.SemaphoreType.DMA,
            pltpu.SemaphoreType.DMA,
        ],
    )
    def k(u_hbm, v_hbm, dst_hbm, src_hbm, h1_hbm,
          id0, is0, id1, is1, ra0, rb0, ra1, rb1,
          semi0, semi1, semr0, semr1, semw0, semw1):
        cid = lax.axis_index("c")
        sid = lax.axis_index("s")
        wid = sid * NC + cid
        nb = jnp.where(wid < rem, nbase + 1, nbase)

        def off(c):
            return pl.multiple_of((wid + c * NW) * ck, 8)

        def issue_idx(c, id_, is_, sem):
            pltpu.async_copy(dst_hbm.at[pl.ds(off(c), ck)], id_, sem)
            pltpu.async_copy(src_hbm.at[pl.ds(off(c), ck)], is_, sem)

        def wait_idx(id_, is_, sem):
            pltpu.make_async_copy(dst_hbm.at[pl.ds(0, ck)], id_, sem).wait()
            pltpu.make_async_copy(src_hbm.at[pl.ds(0, ck)], is_, sem).wait()

        def issue_gather(id_, is_, ra_, rb_, sem):
            pltpu.async_copy(u_hbm.at[id_], ra_, sem)
            pltpu.async_copy(v_hbm.at[is_], rb_, sem)

        def wait_gather(id_, is_, ra_, rb_, sem):
            pltpu.make_async_copy(u_hbm.at[id_], ra_, sem).wait()
            pltpu.make_async_copy(v_hbm.at[is_], rb_, sem).wait()

        def compute(ra_, rb_):
            def rowfn(r, carry2):
                for f in range(hf):
                    sl = pl.ds(f * 16, 16)
                    plsc.addupdate(ra_.at[r, sl], rb_[r, sl])
                return carry2

            lax.fori_loop(0, ck, rowfn, 0)

        def issue_write(c, ra_, sem):
            pltpu.async_copy(ra_, h1_hbm.at[pl.ds(off(c), ck)], sem)

        def wait_write(ra_, sem):
            pltpu.make_async_copy(ra_, h1_hbm.at[pl.ds(0, ck)], sem).wait()

        issue_idx(0, id0, is0, semi0)
        wait_idx(id0, is0, semi0)
        issue_gather(id0, is0, ra0, rb0, semr0)
        issue_idx(1, id1, is1, semi1)

        def pairbody(kp, carry):
            c1 = 2 * kp + 1
            c2 = 2 * kp + 2
            c3 = 2 * kp + 3
            wait_gather(id0, is0, ra0, rb0, semr0)
            compute(ra0, rb0)
            issue_write(2 * kp, ra0, semw0)

            @pl.when(c1 < nb)
            def _():
                wait_idx(id1, is1, semi1)

                @pl.when(kp > 0)
                def _():
                    wait_write(ra1, semw1)

                issue_gather(id1, is1, ra1, rb1, semr1)

                @pl.when(c2 < nb)
                def _():
                    issue_idx(c2, id0, is0, semi0)

            @pl.when(c1 < nb)
            def _():
                wait_gather(id1, is1, ra1, rb1, semr1)
                compute(ra1, rb1)
                issue_write(c1, ra1, semw1)

                @pl.when(c2 < nb)
                def _():
                    wait_idx(id0, is0, semi0)
                    wait_write(ra0, semw0)
                    issue_gather(id0, is0, ra0, rb0, semr0)

                    @pl.when(c3 < nb)
                    def _():
                        issue_idx(c3, id1, is1, semi1)

            return carry

        lax.fori_loop(0, (nb + 1) // 2, pairbody, 0)
        wait_write(ra0, semw0)
        wait_write(ra1, semw1)

    return k(U, V, dst, src)


def _sc_scatter(msg, dst, edge_split):
    ck = 128
    total = N_EDGES // ck
    per_sc = total // 2 if edge_split else total
    ncbase, ncrem = divmod(per_sc, NS)
    w = 128
    zrows = 80
    nblk = N_NODES // zrows

    mesh = plsc.VectorSubcoreMesh(**_SC_MESH)

    @functools.partial(
        pl.kernel,
        mesh=mesh,
        out_type=jax.ShapeDtypeStruct((NC, N_NODES, w), f32),
        scratch_types=[
            pltpu.VMEM((ck,), jnp.int32),
            pltpu.VMEM((ck,), jnp.int32),
            pltpu.VMEM((ck, w), f32),
            pltpu.VMEM((ck, w), f32),
            pltpu.VMEM((zrows, w), f32),
            pltpu.VMEM_SHARED((N_NODES, w), f32),
            pltpu.SemaphoreType.DMA,
            pltpu.SemaphoreType.DMA,
            pltpu.SemaphoreType.DMA,
            pltpu.SemaphoreType.DMA,
        ],
    )
    def k(msg_hbm, dst_hbm, s_hbm, ib0, ib1, bf0, bf1, zbuf, acc_sh,
          semL0, semL1, semS0, semS1):
        cid = lax.axis_index("c")
        sid = lax.axis_index("s")
        zero = jnp.zeros((16,), f32)

        def zf(r, carry):
            for f in range(w // 16):
                zbuf[r, pl.ds(f * 16, 16)] = zero
            return carry

        lax.fori_loop(0, zrows, zf, 0)
        nb = jnp.where(sid < nblk - (nblk // NS) * NS, nblk // NS + 1, nblk // NS)

        def zcopy(j, carry):
            off = pl.multiple_of((sid + j * NS) * zrows, 8)
            pltpu.sync_copy(zbuf, acc_sh.at[pl.ds(off, zrows)])
            return carry

        lax.fori_loop(0, nb, zcopy, 0)
        plsc.subcore_barrier()

        if edge_split:
            base_g = cid * per_sc
        else:
            base_g = 0
        ncb = jnp.where(sid < ncrem, ncbase + 1, ncbase)

        def off(c):
            return pl.multiple_of((base_g + sid + c * NS) * ck, 8)

        def msg_slice(c):
            if edge_split:
                return msg_hbm.at[pl.ds(off(c), ck)]
            return msg_hbm.at[cid, pl.ds(off(c), ck)]

        def issue_load(c, ib, bf, sem):
            pltpu.async_copy(dst_hbm.at[pl.ds(off(c), ck)], ib, sem)
            pltpu.async_copy(msg_slice(c), bf, sem)

        def wait_load(ib, bf, sem):
            pltpu.make_async_copy(dst_hbm.at[pl.ds(0, ck)], ib, sem).wait()
            pltpu.make_async_copy(msg_slice(0), bf, sem).wait()

        def issue_scat(ib, bf, sem):
            pltpu.async_copy(bf, acc_sh.at[ib], sem, add=True)

        def wait_scat(ib, bf, sem):
            pltpu.make_async_copy(bf, acc_sh.at[ib], sem).wait()

        issue_load(0, ib0, bf0, semL0)
        issue_load(1, ib1, bf1, semL1)

        def pairbody(kp, carry):
            c1 = 2 * kp + 1
            c2 = 2 * kp + 2
            c3 = 2 * kp + 3
            wait_load(ib0, bf0, semL0)
            issue_scat(ib0, bf0, semS0)

            @pl.when(c1 < ncb)
            def _():
                wait_load(ib1, bf1, semL1)
                issue_scat(ib1, bf1, semS1)

            @pl.when(c2 < ncb)
            def _():
                wait_scat(ib0, bf0, semS0)
                issue_load(c2, ib0, bf0, semL0)

            @pl.when(c3 < ncb)
            def _():
                wait_scat(ib1, bf1, semS1)
                issue_load(c3, ib1, bf1, semL1)

            return carry

        lax.fori_loop(0, (ncb + 1) // 2, pairbody, 0)
        wait_scat(ib0, bf0, semS0)
        wait_scat(ib1, bf1, semS1)
        plsc.subcore_barrier()

        def outcopy(j, carry):
            off = pl.multiple_of((sid + j * NS) * zrows, 8)
            sl = pl.ds(off, zrows)
            pltpu.sync_copy(acc_sh.at[sl], s_hbm.at[cid, sl])
            return carry

        lax.fori_loop(0, nb, outcopy, 0)

    return k(msg, dst)


def _pad_cols(m, w):
    return jnp.pad(m, ((0, 0), (0, w - m.shape[1])))


def _pad_rows(m, w):
    return jnp.pad(m, ((0, w - m.shape[0]), (0, 0)))


def _split_w1(p):
    w1 = p["W1"]
    d = w1.shape[1] // 2
    wa = w1[:, :d]
    wb = w1[:, d:]
    return (wa - wb).T, wb.T


def _layer_passes(h1, s1, q1, g1, be1, p, w2, wout, cnt_col, split):
    w3 = p["W3"].T
    r = lambda v: v.reshape(1, -1)
    g2, be2 = r(p["g2"]), r(p["be2"])
    g3, be3 = r(p["g3"]), r(p["be3"])
    s2, q2 = _mlp_stats(h1, s1, q1, g1, be1, w2, p["b2"])
    s3, q3 = _mlp2_stats(h1, s1, q1, g1, be1, w2, p["b2"],
                         s2, q2, g2, be2, w3, p["b3"])
    return _mlp2_msg(h1, s1, q1, g1, be1, w2, p["b2"],
                     s2, q2, g2, be2, w3, p["b3"],
                     s3, q3, g3, be3, wout, cnt_col, split)


def kernel(x, edge_index, edge_attr, params):
    del edge_attr
    src = edge_index[0].astype(jnp.int32)
    dst = edge_index[1].astype(jnp.int32)
    p0, p1, p2 = params["net0"], params["net1"], params["net2"]

    wd, wb = _split_w1(p0)
    u, v = _uv_from_x(x, _pad_cols(wd, 128), _pad_cols(wb, 128),
                      _pad_cols(p0["b1"].reshape(1, -1), 128).reshape(-1))
    h1 = _sc_gather(u, v, dst, src)
    s1, q1 = _stats(h1)
    g1p = _pad_cols(p0["g1"].reshape(1, -1), 128)
    be1p = _pad_cols(p0["be1"].reshape(1, -1), 128)
    msg = _layer_passes(h1, s1, q1, g1p, be1p, p0,
                        _pad_rows(p0["W2"].T, 128), 128,
                        cnt_col=True, split=False)
    out0 = _sc_scatter(msg, dst, edge_split=True)

    wd, wb = _split_w1(p1)
    u, v = _uv_sum_halves(out0, out0, wd, wb, p1["b1"], 64)
    h1 = _sc_gather(u, v, dst, src)
    s1, q1 = _stats(h1)
    msg = _layer_passes(h1, s1, q1, p1["g1"].reshape(1, -1),
                        p1["be1"].reshape(1, -1), p1, p1["W2"].T, 128,
                        cnt_col=False, split=False)
    out1 = _sc_scatter(msg, dst, edge_split=True)

    wd, wb = _split_w1(p2)
    u, v = _uv_sum_halves(out1, out0, wd, wb, p2["b1"], 128)
    h1 = _sc_gather(u, v, dst, src)
    s1, q1 = _stats(h1)
    msg2 = _layer_passes(h1, s1, q1, p2["g1"].reshape(1, -1),
                         p2["be1"].reshape(1, -1), p2, p2["W2"].T, 0,
                         cnt_col=False, split=True)
    out2 = _sc_scatter(msg2, dst, edge_split=False)

    return _head(out2, out0, params["lin"]["W"].T, params["lin"]["b"])
